# baseline (speedup 1.0000x reference)
import functools
import jax
import jax.numpy as jnp
from jax import lax
from jax.experimental import pallas as pl
from jax.experimental.pallas import tpu as pltpu
from jax.experimental.pallas import tpu_sc as plsc

_NUM_ENTS = 10000
_NUM_RELS = 230
_HIDDEN = 128
_NUM_EDGES = 160000
_BATCH = 1024

_NC = 2
_NS = 16
_NW = _NC * _NS
_CHUNK = 128
_HALF = _HIDDEN // _NC
_CH_T = 160
_PART = _NS * (_CH_T // 2) * _CHUNK
_ZROW = _NUM_ENTS + _NUM_RELS
_ACC_ROWS = 10112
_RPS = _ACC_ROWS // _NS
_TRASH = _NUM_ENTS

_f32 = jnp.float32
_i32 = jnp.int32


def _sc_rgcn_body(table, gidx, sdst, z64, z16, ones16, acc_out, deg_out,
                  gidx_v, sdst_v, rows0, rows1, zero_v, zero16_v, ones_v,
                  sem0, sem1, sems0, sems1, acc_sh, deg_sh):
    c = lax.axis_index("c")
    s = lax.axis_index("s")
    tbl = table.at[c]

    pltpu.sync_copy(z64, zero_v)
    pltpu.sync_copy(z16, zero16_v)
    pltpu.sync_copy(ones16, ones_v)
    pltpu.sync_copy(gidx.at[s], gidx_v)
    pltpu.sync_copy(sdst.at[s], sdst_v)

    base = s * _RPS
    for off in range(0, _RPS, 128):
        n = min(128, _RPS - off)
        pltpu.sync_copy(zero_v.at[pl.ds(0, n)], acc_sh.at[pl.ds(base + off, n)])
        pltpu.sync_copy(zero16_v.at[pl.ds(0, n)], deg_sh.at[pl.ds(base + off, n)])
    plsc.subcore_barrier()

    half_sel = jnp.where(s < _NS // 2, True, False)

    pltpu.async_copy(tbl.at[gidx_v.at[0]], rows0, sem0)
    pltpu.async_copy(tbl.at[gidx_v.at[1]], rows1, sem1)

    def loop_body(j, carry):
        i0 = 2 * j
        i1 = i0 + 1
        deg_j = jnp.logical_xor(half_sel, j >= _CH_T // 4)

        pltpu.make_async_copy(tbl.at[gidx_v.at[i0]], rows0, sem0).wait()
        pltpu.async_copy(rows0, acc_sh.at[sdst_v.at[i0]], sems0, add=True)

        @pl.when(jnp.logical_and(deg_j, c == 0))
        def _():
            pltpu.sync_copy(ones_v, deg_sh.at[sdst_v.at[i0]], add=True)

        pltpu.make_async_copy(tbl.at[gidx_v.at[i1]], rows1, sem1).wait()
        pltpu.async_copy(rows1, acc_sh.at[sdst_v.at[i1]], sems1, add=True)

        @pl.when(jnp.logical_and(deg_j, c == 1))
        def _():
            pltpu.sync_copy(ones_v, deg_sh.at[sdst_v.at[i1]], add=True)

        @pl.when(j < _CH_T // 2 - 1)
        def _():
            pltpu.make_async_copy(rows0, acc_sh.at[sdst_v.at[i0]], sems0).wait()
            pltpu.async_copy(tbl.at[gidx_v.at[i0 + 2]], rows0, sem0)
            pltpu.make_async_copy(rows1, acc_sh.at[sdst_v.at[i1]], sems1).wait()
            pltpu.async_copy(tbl.at[gidx_v.at[i1 + 2]], rows1, sem1)

        return carry

    lax.fori_loop(0, _CH_T // 2, loop_body, 0)
    pltpu.make_async_copy(rows0, acc_sh.at[sdst_v.at[_CH_T - 2]], sems0).wait()
    pltpu.make_async_copy(rows1, acc_sh.at[sdst_v.at[_CH_T - 1]], sems1).wait()
    plsc.subcore_barrier()
    pltpu.sync_copy(acc_sh.at[pl.ds(base, _RPS)], acc_out.at[c, pl.ds(base, _RPS)])
    pltpu.sync_copy(deg_sh.at[pl.ds(base, _RPS)], deg_out.at[c, pl.ds(base, _RPS)])


@functools.cache
def _get_sc_rgcn():
    return pl.kernel(
        _sc_rgcn_body,
        out_type=[
            jax.ShapeDtypeStruct((_NC, _ACC_ROWS, _HALF), _f32),
            jax.ShapeDtypeStruct((_NC, _ACC_ROWS, 16), _f32),
        ],
        mesh=plsc.VectorSubcoreMesh(core_axis_name="c", subcore_axis_name="s"),
        compiler_params=pltpu.CompilerParams(use_tc_tiling_on_sc=False),
        scratch_types=[
            pltpu.VMEM((_CH_T, _CHUNK), _i32),
            pltpu.VMEM((_CH_T, _CHUNK), _i32),
            pltpu.VMEM((_CHUNK, _HALF), _f32),
            pltpu.VMEM((_CHUNK, _HALF), _f32),
            pltpu.VMEM((128, _HALF), _f32),
            pltpu.VMEM((128, 16), _f32),
            pltpu.VMEM((_CHUNK, 16), _f32),
            pltpu.SemaphoreType.DMA,
            pltpu.SemaphoreType.DMA,
            pltpu.SemaphoreType.DMA,
            pltpu.SemaphoreType.DMA,
            pltpu.VMEM_SHARED((_ACC_ROWS, _HALF), _f32),
            pltpu.VMEM_SHARED((_ACC_ROWS, 16), _f32),
        ],
    )


_GPW = 3 * _BATCH // _NW


def _sc_gather_body(table, idx, out, idx_v, rows_v, sem):
    c = lax.axis_index("c")
    s = lax.axis_index("s")
    wid = c * _NS + s
    pltpu.sync_copy(idx.at[wid], idx_v)
    pltpu.async_copy(table.at[idx_v], rows_v, sem).wait()
    pltpu.sync_copy(rows_v, out.at[pl.ds(wid * _GPW, _GPW)])


@functools.cache
def _get_sc_gather():
    return pl.kernel(
        _sc_gather_body,
        out_type=jax.ShapeDtypeStruct((3 * _BATCH, _HIDDEN), _f32),
        mesh=plsc.VectorSubcoreMesh(core_axis_name="c", subcore_axis_name="s"),
        scratch_types=[
            pltpu.VMEM((_GPW,), _i32),
            pltpu.VMEM((_GPW, _HIDDEN), _f32),
            pltpu.SemaphoreType.DMA,
        ],
    )


def _mm_body(x_ref, w_ref, o_ref):
    o_ref[...] = jnp.dot(x_ref[...], w_ref[...], preferred_element_type=_f32)


def _mm(x, w, bm):
    m, k = x.shape
    n = w.shape[1]
    return pl.pallas_call(
        _mm_body,
        grid=(m // bm,),
        in_specs=[
            pl.BlockSpec((bm, k), lambda i: (i, 0)),
            pl.BlockSpec((k, n), lambda i: (0, 0)),
        ],
        out_specs=pl.BlockSpec((bm, n), lambda i: (i, 0)),
        out_shape=jax.ShapeDtypeStruct((m, n), _f32),
    )(x, w)


def _fin_body(a0_ref, a1_ref, d0_ref, d1_ref, hs_ref, o_ref):
    deg = jnp.maximum(d0_ref[...][:, 0:1] + d1_ref[...][:, 0:1], 1.0)
    agg = jnp.concatenate([a0_ref[...], a1_ref[...]], axis=1) / deg
    o_ref[...] = jnp.maximum(agg + hs_ref[...], 0.0)


def _finalize(a0, a1, d0, d1, hs, bm):
    m = a0.shape[0]
    return pl.pallas_call(
        _fin_body,
        grid=(m // bm,),
        in_specs=[
            pl.BlockSpec((bm, _HALF), lambda i: (i, 0)),
            pl.BlockSpec((bm, _HALF), lambda i: (i, 0)),
            pl.BlockSpec((bm, 16), lambda i: (i, 0)),
            pl.BlockSpec((bm, 16), lambda i: (i, 0)),
            pl.BlockSpec((bm, _HIDDEN), lambda i: (i, 0)),
        ],
        out_specs=pl.BlockSpec((bm, _HIDDEN), lambda i: (i, 0)),
        out_shape=jax.ShapeDtypeStruct((m, _HIDDEN), _f32),
    )(a0, a1, d0, d1, hs)


def _gru_body(x_ref, h_ref, wih_ref, whh_ref, bih_ref, bhh_ref, o_ref):
    x = x_ref[...]
    h = h_ref[...]
    gi = jnp.dot(x, wih_ref[...], preferred_element_type=_f32) + bih_ref[...]
    gh = jnp.dot(h, whh_ref[...], preferred_element_type=_f32) + bhh_ref[...]
    hd = _HIDDEN
    r = jax.nn.sigmoid(gi[:, :hd] + gh[:, :hd])
    z = jax.nn.sigmoid(gi[:, hd:2 * hd] + gh[:, hd:2 * hd])
    n = jnp.tanh(gi[:, 2 * hd:] + r * gh[:, 2 * hd:])
    hn = (1.0 - z) * n + z * h
    norm = jnp.sqrt(jnp.sum(hn * hn, axis=1, keepdims=True))
    o_ref[...] = hn / jnp.maximum(norm, 1e-12)


def _gru(x, h, wih_t, whh_t, bih, bhh):
    m = x.shape[0]
    return pl.pallas_call(
        _gru_body,
        in_specs=[pl.BlockSpec(a.shape, lambda: (0,) * a.ndim)
                  for a in (x, h, wih_t, whh_t, bih, bhh)],
        out_specs=pl.BlockSpec((m, _HIDDEN), lambda: (0, 0)),
        out_shape=jax.ShapeDtypeStruct((m, _HIDDEN), _f32),
    )(x, h, wih_t, whh_t, bih, bhh)


def _dec_body(e1_ref, e2_ref, w_ref, b_ref, fcw_ref, fcb_ref, o_ref, *, bm):
    e1 = e1_ref[...]
    e2 = e2_ref[...]
    z = jnp.zeros((bm, 1), _f32)
    shifts = []
    for e in (e1, e2):
        shifts.append(jnp.concatenate([z, e[:, :-1]], axis=1))
        shifts.append(e)
        shifts.append(jnp.concatenate([e[:, 1:], z], axis=1))
    acc = jnp.zeros((bm, _HIDDEN), _f32)
    for ch in range(50):
        conv = b_ref[ch]
        for j in range(6):
            conv = conv + shifts[j] * w_ref[6 * ch + j]
        conv = jnp.maximum(conv, 0.0)
        acc = acc + jnp.dot(conv, fcw_ref[ch], preferred_element_type=_f32)
    o_ref[...] = jnp.maximum(acc + fcb_ref[...], 0.0)


def _decoder(e1, e2, conv_w, conv_b, fc_w, fc_b, bm=256):
    m = e1.shape[0]
    wflat = conv_w.reshape(300)
    fcw3 = fc_w.reshape(50, _HIDDEN, _HIDDEN)
    fcb = fc_b.reshape(1, _HIDDEN)
    return pl.pallas_call(
        functools.partial(_dec_body, bm=bm),
        grid=(m // bm,),
        in_specs=[
            pl.BlockSpec((bm, _HIDDEN), lambda i: (i, 0)),
            pl.BlockSpec((bm, _HIDDEN), lambda i: (i, 0)),
            pl.BlockSpec(memory_space=pltpu.SMEM),
            pl.BlockSpec(memory_space=pltpu.SMEM),
            pl.BlockSpec((50, _HIDDEN, _HIDDEN), lambda i: (0, 0, 0)),
            pl.BlockSpec((1, _HIDDEN), lambda i: (0, 0)),
        ],
        out_specs=pl.BlockSpec((bm, _HIDDEN), lambda i: (i, 0)),
        out_shape=jax.ShapeDtypeStruct((m, _HIDDEN), _f32),
    )(e1, e2, wflat, conv_b, fcw3, fcb)


def _score_body(y_ref, t_ref, o_ref):
    o_ref[...] = lax.dot_general(
        y_ref[...], t_ref[...], (((1,), (1,)), ((), ())),
        preferred_element_type=_f32)


def _score(y, table, bn):
    m = y.shape[0]
    n = table.shape[0]
    return pl.pallas_call(
        _score_body,
        grid=(n // bn,),
        in_specs=[
            pl.BlockSpec((m, _HIDDEN), lambda i: (0, 0)),
            pl.BlockSpec((bn, _HIDDEN), lambda i: (i, 0)),
        ],
        out_specs=pl.BlockSpec((m, bn), lambda i: (0, i)),
        out_shape=jax.ShapeDtypeStruct((m, n), _f32),
    )(y, table)


def _pad_to(x, n, val):
    return jnp.concatenate(
        [x, jnp.full((n - x.shape[0],), val, dtype=x.dtype)])


@jax.jit
def kernel(ent_embeds, rel_embeds, time_embeds, rgcn_w_msg, rgcn_w_self,
           gru_w_ih, gru_w_hh, gru_b_ih, gru_b_hh, conve_w, conve_b,
           conve_fc_w, conve_fc_b, convr_w, convr_b, convr_fc_w, convr_fc_b,
           edge_src, edge_dst, edge_type, subj, rel, obj):
    hist = time_embeds.shape[0]
    nlayers = rgcn_w_msg.shape[0]

    src = edge_src.astype(_i32)
    dst = edge_dst.astype(_i32)
    ety = edge_type.astype(_i32)

    gidx_ts, sdst_ts = [], []
    for t in range(hist):
        ga = _pad_to(src[t], _PART, _ZROW)
        gb = _pad_to(ety[t] + _NUM_ENTS, _PART, _ZROW)
        da = _pad_to(dst[t], _PART, _TRASH)
        gidx_ts.append(jnp.concatenate([ga, gb]).reshape(_NS, _CH_T, _CHUNK))
        sdst_ts.append(jnp.concatenate([da, da]).reshape(_NS, _CH_T, _CHUNK))

    z64 = jnp.zeros((128, _HALF), _f32)
    z16 = jnp.zeros((128, 16), _f32)
    ones16 = jnp.ones((_CHUNK, 16), _f32)
    zrow = jnp.zeros((1, _HIDDEN), _f32)

    wih_t = gru_w_ih.T
    whh_t = gru_w_hh.T
    bih = gru_b_ih.reshape(1, -1)
    bhh = gru_b_hh.reshape(1, -1)
    relpad = jnp.zeros((232 - _NUM_RELS, _HIDDEN), _f32)
    rel_p = jnp.concatenate([rel_embeds, relpad])

    ent = ent_embeds
    relh_p = rel_p
    for t in range(hist):
        for l in range(nlayers):
            w2 = jnp.concatenate([rgcn_w_msg[l], rgcn_w_self[l]], axis=1)
            hw_hs = _mm(ent, w2, bm=2000)
            rw = _mm(relh_p, rgcn_w_msg[l], bm=232)[:_NUM_RELS]
            table = jnp.concatenate([hw_hs[:, :_HIDDEN], rw, zrow])
            table_h = table.reshape(_ZROW + 1, _NC, _HALF).transpose(1, 0, 2)
            acc, degp = _get_sc_rgcn()(table_h, gidx_ts[t], sdst_ts[t],
                                       z64, z16, ones16)
            ent = _finalize(acc[0, :_NUM_ENTS], acc[1, :_NUM_ENTS],
                            degp[0, :_NUM_ENTS], degp[1, :_NUM_ENTS],
                            hw_hs[:, _HIDDEN:], bm=2000)
        relh_p = _gru(rel_p, relh_p, wih_t, whh_t, bih, bhh)

    relh = relh_p[:_NUM_RELS]

    table2 = jnp.concatenate([ent, relh])
    idx = jnp.concatenate([subj.astype(_i32), obj.astype(_i32),
                           rel.astype(_i32) + _NUM_ENTS]).reshape(_NW, _GPW)
    rows = _get_sc_gather()(table2, idx)
    e_subj = rows[:_BATCH]
    e_obj = rows[_BATCH:2 * _BATCH]
    e_rel = rows[2 * _BATCH:]

    y1 = _decoder(e_subj, e_rel, conve_w, conve_b, conve_fc_w, conve_fc_b)
    y2 = _decoder(e_subj, e_obj, convr_w, convr_b, convr_fc_w, convr_fc_b)

    ent10240 = jnp.concatenate([ent, jnp.zeros((240, _HIDDEN), _f32)])
    ent_logit = _score(y1, ent10240, bn=2048)[:, :_NUM_ENTS]
    relh256 = jnp.concatenate([relh, jnp.zeros((26, _HIDDEN), _f32)])
    rel_logit = _score(y2, relh256, bn=256)[:, :_NUM_RELS]
    return ent_logit, rel_logit


# speedup vs baseline: 1.1376x; 1.1376x over previous
import functools
import jax
import jax.numpy as jnp
from jax import lax
from jax.experimental import pallas as pl
from jax.experimental.pallas import tpu as pltpu
from jax.experimental.pallas import tpu_sc as plsc

_NUM_ENTS = 10000
_NUM_RELS = 230
_HIDDEN = 128
_NUM_EDGES = 160000
_BATCH = 1024

_NC = 2
_NS = 16
_NW = _NC * _NS
_CHUNK = 128
_HALF = _HIDDEN // _NC
_CH_T = 160
_PART = _NS * (_CH_T // 2) * _CHUNK
_ZROW = _NUM_ENTS + _NUM_RELS
_ACC_ROWS = 10112
_RPS = _ACC_ROWS // _NS
_TRASH = _NUM_ENTS

_f32 = jnp.float32
_i32 = jnp.int32


def _edge_loop(src, c, deg_lo, gidx_v, sdst_v, rows0, rows1, ones_v,
               sem0, sem1, acc_sh, deg_sh):
    pltpu.async_copy(src.at[gidx_v.at[0]], rows0, sem0)

    def loop_body(j, carry):
        i0 = 2 * j
        i1 = i0 + 1
        deg_j = (j < _CH_T // 4) if deg_lo else (j >= _CH_T // 4)

        pltpu.make_async_copy(src.at[gidx_v.at[i0]], rows0, sem0).wait()
        pltpu.async_copy(src.at[gidx_v.at[i1]], rows1, sem1)
        pltpu.sync_copy(rows0, acc_sh.at[sdst_v.at[i0]], add=True)

        @pl.when(jnp.logical_and(deg_j, c == 0))
        def _():
            pltpu.sync_copy(ones_v, deg_sh.at[sdst_v.at[i0]], add=True)

        pltpu.make_async_copy(src.at[gidx_v.at[i1]], rows1, sem1).wait()

        @pl.when(j < _CH_T // 2 - 1)
        def _():
            pltpu.async_copy(src.at[gidx_v.at[i0 + 2]], rows0, sem0)

        pltpu.sync_copy(rows1, acc_sh.at[sdst_v.at[i1]], add=True)

        @pl.when(jnp.logical_and(deg_j, c == 1))
        def _():
            pltpu.sync_copy(ones_v, deg_sh.at[sdst_v.at[i1]], add=True)

        return carry

    lax.fori_loop(0, _CH_T // 2, loop_body, 0)


def _sc_rgcn_body(table, rw, gidx, sdst, z64, z16, ones16, acc_out, deg_out,
                  gidx_v, sdst_v, rows0, rows1, zero_v, zero16_v, ones_v,
                  sem0, sem1, acc_sh, deg_sh, rw_sh):
    c = lax.axis_index("c")
    s = lax.axis_index("s")
    tbl = table.at[c]

    pltpu.sync_copy(z64, zero_v)
    pltpu.sync_copy(z16, zero16_v)
    pltpu.sync_copy(ones16, ones_v)
    pltpu.sync_copy(gidx.at[s], gidx_v)
    pltpu.sync_copy(sdst.at[s], sdst_v)

    @pl.when(s == 0)
    def _():
        pltpu.sync_copy(rw.at[c], rw_sh)

    base = s * _RPS
    for off in range(0, _RPS, 128):
        n = min(128, _RPS - off)
        pltpu.sync_copy(zero_v.at[pl.ds(0, n)], acc_sh.at[pl.ds(base + off, n)])
        pltpu.sync_copy(zero16_v.at[pl.ds(0, n)], deg_sh.at[pl.ds(base + off, n)])
    plsc.subcore_barrier()

    @pl.when(s < _NS // 2)
    def _():
        _edge_loop(tbl, c, True, gidx_v, sdst_v, rows0, rows1, ones_v,
                   sem0, sem1, acc_sh, deg_sh)

    @pl.when(s >= _NS // 2)
    def _():
        _edge_loop(rw_sh, c, False, gidx_v, sdst_v, rows0, rows1, ones_v,
                   sem0, sem1, acc_sh, deg_sh)

    plsc.subcore_barrier()
    pltpu.sync_copy(acc_sh.at[pl.ds(base, _RPS)], acc_out.at[c, pl.ds(base, _RPS)])
    pltpu.sync_copy(deg_sh.at[pl.ds(base, _RPS)], deg_out.at[c, pl.ds(base, _RPS)])


@functools.cache
def _get_sc_rgcn():
    return pl.kernel(
        _sc_rgcn_body,
        out_type=[
            jax.ShapeDtypeStruct((_NC, _ACC_ROWS, _HALF), _f32),
            jax.ShapeDtypeStruct((_NC, _ACC_ROWS, 16), _f32),
        ],
        mesh=plsc.VectorSubcoreMesh(core_axis_name="c", subcore_axis_name="s"),
        compiler_params=pltpu.CompilerParams(use_tc_tiling_on_sc=False),
        scratch_types=[
            pltpu.VMEM((_CH_T, _CHUNK), _i32),
            pltpu.VMEM((_CH_T, _CHUNK), _i32),
            pltpu.VMEM((_CHUNK, _HALF), _f32),
            pltpu.VMEM((_CHUNK, _HALF), _f32),
            pltpu.VMEM((128, _HALF), _f32),
            pltpu.VMEM((128, 16), _f32),
            pltpu.VMEM((_CHUNK, 16), _f32),
            pltpu.SemaphoreType.DMA,
            pltpu.SemaphoreType.DMA,
            pltpu.VMEM_SHARED((_ACC_ROWS, _HALF), _f32),
            pltpu.VMEM_SHARED((_ACC_ROWS, 16), _f32),
            pltpu.VMEM_SHARED((_NUM_RELS + 1, _HALF), _f32),
        ],
    )


_GPW = 3 * _BATCH // _NW


def _sc_gather_body(table, idx, out, idx_v, rows_v, sem):
    c = lax.axis_index("c")
    s = lax.axis_index("s")
    wid = c * _NS + s
    pltpu.sync_copy(idx.at[wid], idx_v)
    pltpu.async_copy(table.at[idx_v], rows_v, sem).wait()
    pltpu.sync_copy(rows_v, out.at[pl.ds(wid * _GPW, _GPW)])


@functools.cache
def _get_sc_gather():
    return pl.kernel(
        _sc_gather_body,
        out_type=jax.ShapeDtypeStruct((3 * _BATCH, _HIDDEN), _f32),
        mesh=plsc.VectorSubcoreMesh(core_axis_name="c", subcore_axis_name="s"),
        scratch_types=[
            pltpu.VMEM((_GPW,), _i32),
            pltpu.VMEM((_GPW, _HIDDEN), _f32),
            pltpu.SemaphoreType.DMA,
        ],
    )


def _mm_body(x_ref, w_ref, o_ref):
    o_ref[...] = jnp.dot(x_ref[...], w_ref[...], preferred_element_type=_f32)


def _mm(x, w, bm):
    m, k = x.shape
    n = w.shape[1]
    return pl.pallas_call(
        _mm_body,
        grid=(m // bm,),
        in_specs=[
            pl.BlockSpec((bm, k), lambda i: (i, 0)),
            pl.BlockSpec((k, n), lambda i: (0, 0)),
        ],
        out_specs=pl.BlockSpec((bm, n), lambda i: (i, 0)),
        out_shape=jax.ShapeDtypeStruct((m, n), _f32),
    )(x, w)


def _fin_body(a0_ref, a1_ref, d0_ref, d1_ref, hs_ref, o_ref):
    deg = jnp.maximum(d0_ref[...][:, 0:1] + d1_ref[...][:, 0:1], 1.0)
    agg = jnp.concatenate([a0_ref[...], a1_ref[...]], axis=1) / deg
    o_ref[...] = jnp.maximum(agg + hs_ref[...], 0.0)


def _finalize(a0, a1, d0, d1, hs, bm):
    m = a0.shape[0]
    return pl.pallas_call(
        _fin_body,
        grid=(m // bm,),
        in_specs=[
            pl.BlockSpec((bm, _HALF), lambda i: (i, 0)),
            pl.BlockSpec((bm, _HALF), lambda i: (i, 0)),
            pl.BlockSpec((bm, 16), lambda i: (i, 0)),
            pl.BlockSpec((bm, 16), lambda i: (i, 0)),
            pl.BlockSpec((bm, _HIDDEN), lambda i: (i, 0)),
        ],
        out_specs=pl.BlockSpec((bm, _HIDDEN), lambda i: (i, 0)),
        out_shape=jax.ShapeDtypeStruct((m, _HIDDEN), _f32),
    )(a0, a1, d0, d1, hs)


def _gru_body(x_ref, h_ref, wih_ref, whh_ref, bih_ref, bhh_ref, o_ref):
    x = x_ref[...]
    h = h_ref[...]
    gi = jnp.dot(x, wih_ref[...], preferred_element_type=_f32) + bih_ref[...]
    gh = jnp.dot(h, whh_ref[...], preferred_element_type=_f32) + bhh_ref[...]
    hd = _HIDDEN
    r = jax.nn.sigmoid(gi[:, :hd] + gh[:, :hd])
    z = jax.nn.sigmoid(gi[:, hd:2 * hd] + gh[:, hd:2 * hd])
    n = jnp.tanh(gi[:, 2 * hd:] + r * gh[:, 2 * hd:])
    hn = (1.0 - z) * n + z * h
    norm = jnp.sqrt(jnp.sum(hn * hn, axis=1, keepdims=True))
    o_ref[...] = hn / jnp.maximum(norm, 1e-12)


def _gru(x, h, wih_t, whh_t, bih, bhh):
    m = x.shape[0]
    return pl.pallas_call(
        _gru_body,
        in_specs=[pl.BlockSpec(a.shape, lambda: (0,) * a.ndim)
                  for a in (x, h, wih_t, whh_t, bih, bhh)],
        out_specs=pl.BlockSpec((m, _HIDDEN), lambda: (0, 0)),
        out_shape=jax.ShapeDtypeStruct((m, _HIDDEN), _f32),
    )(x, h, wih_t, whh_t, bih, bhh)


def _dec_body(e1_ref, e2_ref, w_ref, b_ref, fcw_ref, fcb_ref, o_ref, *, bm):
    e1 = e1_ref[...]
    e2 = e2_ref[...]
    z = jnp.zeros((bm, 1), _f32)
    shifts = []
    for e in (e1, e2):
        shifts.append(jnp.concatenate([z, e[:, :-1]], axis=1))
        shifts.append(e)
        shifts.append(jnp.concatenate([e[:, 1:], z], axis=1))
    acc = jnp.zeros((bm, _HIDDEN), _f32)
    for ch in range(50):
        conv = b_ref[ch]
        for j in range(6):
            conv = conv + shifts[j] * w_ref[6 * ch + j]
        conv = jnp.maximum(conv, 0.0)
        acc = acc + jnp.dot(conv, fcw_ref[ch], preferred_element_type=_f32)
    o_ref[...] = jnp.maximum(acc + fcb_ref[...], 0.0)


def _decoder(e1, e2, conv_w, conv_b, fc_w, fc_b, bm=256):
    m = e1.shape[0]
    wflat = conv_w.reshape(300)
    fcw3 = fc_w.reshape(50, _HIDDEN, _HIDDEN)
    fcb = fc_b.reshape(1, _HIDDEN)
    return pl.pallas_call(
        functools.partial(_dec_body, bm=bm),
        grid=(m // bm,),
        in_specs=[
            pl.BlockSpec((bm, _HIDDEN), lambda i: (i, 0)),
            pl.BlockSpec((bm, _HIDDEN), lambda i: (i, 0)),
            pl.BlockSpec(memory_space=pltpu.SMEM),
            pl.BlockSpec(memory_space=pltpu.SMEM),
            pl.BlockSpec((50, _HIDDEN, _HIDDEN), lambda i: (0, 0, 0)),
            pl.BlockSpec((1, _HIDDEN), lambda i: (0, 0)),
        ],
        out_specs=pl.BlockSpec((bm, _HIDDEN), lambda i: (i, 0)),
        out_shape=jax.ShapeDtypeStruct((m, _HIDDEN), _f32),
    )(e1, e2, wflat, conv_b, fcw3, fcb)


def _score_body(y_ref, t_ref, o_ref):
    o_ref[...] = lax.dot_general(
        y_ref[...], t_ref[...], (((1,), (1,)), ((), ())),
        preferred_element_type=_f32)


def _score(y, table, bn):
    m = y.shape[0]
    n = table.shape[0]
    return pl.pallas_call(
        _score_body,
        grid=(n // bn,),
        in_specs=[
            pl.BlockSpec((m, _HIDDEN), lambda i: (0, 0)),
            pl.BlockSpec((bn, _HIDDEN), lambda i: (i, 0)),
        ],
        out_specs=pl.BlockSpec((m, bn), lambda i: (0, i)),
        out_shape=jax.ShapeDtypeStruct((m, n), _f32),
    )(y, table)


def _pad_to(x, n, val):
    return jnp.concatenate(
        [x, jnp.full((n - x.shape[0],), val, dtype=x.dtype)])


@jax.jit
def kernel(ent_embeds, rel_embeds, time_embeds, rgcn_w_msg, rgcn_w_self,
           gru_w_ih, gru_w_hh, gru_b_ih, gru_b_hh, conve_w, conve_b,
           conve_fc_w, conve_fc_b, convr_w, convr_b, convr_fc_w, convr_fc_b,
           edge_src, edge_dst, edge_type, subj, rel, obj):
    hist = time_embeds.shape[0]
    nlayers = rgcn_w_msg.shape[0]

    src = edge_src.astype(_i32)
    dst = edge_dst.astype(_i32)
    ety = edge_type.astype(_i32)

    gidx_ts, sdst_ts = [], []
    for t in range(hist):
        ga = _pad_to(src[t], _PART, _NUM_ENTS)
        gb = _pad_to(ety[t], _PART, _NUM_RELS)
        da = _pad_to(dst[t], _PART, _TRASH)
        gidx_ts.append(jnp.concatenate([ga, gb]).reshape(_NS, _CH_T, _CHUNK))
        sdst_ts.append(jnp.concatenate([da, da]).reshape(_NS, _CH_T, _CHUNK))

    z64 = jnp.zeros((128, _HALF), _f32)
    z16 = jnp.zeros((128, 16), _f32)
    ones16 = jnp.ones((_CHUNK, 16), _f32)
    zrow = jnp.zeros((1, _HIDDEN), _f32)

    wih_t = gru_w_ih.T
    whh_t = gru_w_hh.T
    bih = gru_b_ih.reshape(1, -1)
    bhh = gru_b_hh.reshape(1, -1)
    relpad = jnp.zeros((232 - _NUM_RELS, _HIDDEN), _f32)
    rel_p = jnp.concatenate([rel_embeds, relpad])

    ent = ent_embeds
    relh_p = rel_p
    for t in range(hist):
        for l in range(nlayers):
            w2 = jnp.concatenate([rgcn_w_msg[l], rgcn_w_self[l]], axis=1)
            hw_hs = _mm(ent, w2, bm=2000)
            rw = _mm(relh_p, rgcn_w_msg[l], bm=232)[:_NUM_RELS]
            table = jnp.concatenate([hw_hs[:, :_HIDDEN], zrow])
            table_h = table.reshape(_NUM_ENTS + 1, _NC, _HALF).transpose(1, 0, 2)
            rw_h = jnp.concatenate([rw, zrow]).reshape(
                _NUM_RELS + 1, _NC, _HALF).transpose(1, 0, 2)
            acc, degp = _get_sc_rgcn()(table_h, rw_h, gidx_ts[t], sdst_ts[t],
                                       z64, z16, ones16)
            ent = _finalize(acc[0, :_NUM_ENTS], acc[1, :_NUM_ENTS],
                            degp[0, :_NUM_ENTS], degp[1, :_NUM_ENTS],
                            hw_hs[:, _HIDDEN:], bm=2000)
        relh_p = _gru(rel_p, relh_p, wih_t, whh_t, bih, bhh)

    relh = relh_p[:_NUM_RELS]

    table2 = jnp.concatenate([ent, relh])
    idx = jnp.concatenate([subj.astype(_i32), obj.astype(_i32),
                           rel.astype(_i32) + _NUM_ENTS]).reshape(_NW, _GPW)
    rows = _get_sc_gather()(table2, idx)
    e_subj = rows[:_BATCH]
    e_obj = rows[_BATCH:2 * _BATCH]
    e_rel = rows[2 * _BATCH:]

    y1 = _decoder(e_subj, e_rel, conve_w, conve_b, conve_fc_w, conve_fc_b)
    y2 = _decoder(e_subj, e_obj, convr_w, convr_b, convr_fc_w, convr_fc_b)

    ent10240 = jnp.concatenate([ent, jnp.zeros((240, _HIDDEN), _f32)])
    ent_logit = _score(y1, ent10240, bn=2048)[:, :_NUM_ENTS]
    relh256 = jnp.concatenate([relh, jnp.zeros((26, _HIDDEN), _f32)])
    rel_logit = _score(y2, relh256, bn=256)[:, :_NUM_RELS]
    return ent_logit, rel_logit


# speedup vs baseline: 1.1862x; 1.0428x over previous
import functools
import jax
import jax.numpy as jnp
from jax import lax
from jax.experimental import pallas as pl
from jax.experimental.pallas import tpu as pltpu
from jax.experimental.pallas import tpu_sc as plsc

_NUM_ENTS = 10000
_NUM_RELS = 230
_HIDDEN = 128
_NUM_EDGES = 160000
_BATCH = 1024

_NC = 2
_NS = 16
_NW = _NC * _NS
_CHUNK = 128
_HALF = _HIDDEN // _NC
_CH_T = 160
_PART = _NS * (_CH_T // 2) * _CHUNK
_ZROW = _NUM_ENTS + _NUM_RELS
_ACC_ROWS = 10112
_RPS = _ACC_ROWS // _NS
_TRASH = _NUM_ENTS

_f32 = jnp.float32
_i32 = jnp.int32


def _edge_loop(tbl, rw_sh, c, gidx_v, sdst_v, rows0, rows1, ones_v,
               sem0, sem1, acc_sh, deg_sh):
    nb = _CH_T // 2
    pltpu.async_copy(tbl.at[gidx_v.at[0]], rows0, sem0)

    def loop_body(j, carry):
        jb = j + nb

        pltpu.make_async_copy(tbl.at[gidx_v.at[j]], rows0, sem0).wait()
        pltpu.async_copy(rw_sh.at[gidx_v.at[jb]], rows1, sem1)
        pltpu.sync_copy(rows0, acc_sh.at[sdst_v.at[j]], add=True)

        @pl.when(lax.rem(j, 2) == c)
        def _():
            pltpu.sync_copy(ones_v, deg_sh.at[sdst_v.at[j]], add=True)

        pltpu.make_async_copy(rw_sh.at[gidx_v.at[jb]], rows1, sem1).wait()

        @pl.when(j < nb - 1)
        def _():
            pltpu.async_copy(tbl.at[gidx_v.at[j + 1]], rows0, sem0)

        pltpu.sync_copy(rows1, acc_sh.at[sdst_v.at[jb]], add=True)
        return carry

    lax.fori_loop(0, nb, loop_body, 0)


def _sc_rgcn_body(table, rw, gidx, sdst, z64, z16, ones16, acc_out, deg_out,
                  gidx_v, sdst_v, rows0, rows1, zero_v, zero16_v, ones_v,
                  sem0, sem1, acc_sh, deg_sh, rw_sh):
    c = lax.axis_index("c")
    s = lax.axis_index("s")
    tbl = table.at[c]

    pltpu.sync_copy(z64, zero_v)
    pltpu.sync_copy(z16, zero16_v)
    pltpu.sync_copy(ones16, ones_v)
    pltpu.sync_copy(gidx.at[s], gidx_v)
    pltpu.sync_copy(sdst.at[s], sdst_v)

    @pl.when(s == 0)
    def _():
        pltpu.sync_copy(rw.at[c], rw_sh)

    base = s * _RPS
    for off in range(0, _RPS, 128):
        n = min(128, _RPS - off)
        pltpu.sync_copy(zero_v.at[pl.ds(0, n)], acc_sh.at[pl.ds(base + off, n)])
        pltpu.sync_copy(zero16_v.at[pl.ds(0, n)], deg_sh.at[pl.ds(base + off, n)])
    plsc.subcore_barrier()

    _edge_loop(tbl, rw_sh, c, gidx_v, sdst_v, rows0, rows1, ones_v,
               sem0, sem1, acc_sh, deg_sh)
    plsc.subcore_barrier()
    pltpu.sync_copy(acc_sh.at[pl.ds(base, _RPS)], acc_out.at[c, pl.ds(base, _RPS)])
    pltpu.sync_copy(deg_sh.at[pl.ds(base, _RPS)], deg_out.at[c, pl.ds(base, _RPS)])


@functools.cache
def _get_sc_rgcn():
    return pl.kernel(
        _sc_rgcn_body,
        out_type=[
            jax.ShapeDtypeStruct((_NC, _ACC_ROWS, _HALF), _f32),
            jax.ShapeDtypeStruct((_NC, _ACC_ROWS, 16), _f32),
        ],
        mesh=plsc.VectorSubcoreMesh(core_axis_name="c", subcore_axis_name="s"),
        compiler_params=pltpu.CompilerParams(use_tc_tiling_on_sc=False),
        scratch_types=[
            pltpu.VMEM((_CH_T, _CHUNK), _i32),
            pltpu.VMEM((_CH_T, _CHUNK), _i32),
            pltpu.VMEM((_CHUNK, _HALF), _f32),
            pltpu.VMEM((_CHUNK, _HALF), _f32),
            pltpu.VMEM((128, _HALF), _f32),
            pltpu.VMEM((128, 16), _f32),
            pltpu.VMEM((_CHUNK, 16), _f32),
            pltpu.SemaphoreType.DMA,
            pltpu.SemaphoreType.DMA,
            pltpu.VMEM_SHARED((_ACC_ROWS, _HALF), _f32),
            pltpu.VMEM_SHARED((_ACC_ROWS, 16), _f32),
            pltpu.VMEM_SHARED((_NUM_RELS + 1, _HALF), _f32),
        ],
    )


_GPW = 3 * _BATCH // _NW


def _sc_gather_body(table, idx, out, idx_v, rows_v, sem):
    c = lax.axis_index("c")
    s = lax.axis_index("s")
    wid = c * _NS + s
    pltpu.sync_copy(idx.at[wid], idx_v)
    pltpu.async_copy(table.at[idx_v], rows_v, sem).wait()
    pltpu.sync_copy(rows_v, out.at[pl.ds(wid * _GPW, _GPW)])


@functools.cache
def _get_sc_gather():
    return pl.kernel(
        _sc_gather_body,
        out_type=jax.ShapeDtypeStruct((3 * _BATCH, _HIDDEN), _f32),
        mesh=plsc.VectorSubcoreMesh(core_axis_name="c", subcore_axis_name="s"),
        scratch_types=[
            pltpu.VMEM((_GPW,), _i32),
            pltpu.VMEM((_GPW, _HIDDEN), _f32),
            pltpu.SemaphoreType.DMA,
        ],
    )


def _mm_body(x_ref, w_ref, o_ref):
    o_ref[...] = jnp.dot(x_ref[...], w_ref[...], preferred_element_type=_f32)


def _mm(x, w, bm):
    m, k = x.shape
    n = w.shape[1]
    return pl.pallas_call(
        _mm_body,
        grid=(m // bm,),
        in_specs=[
            pl.BlockSpec((bm, k), lambda i: (i, 0)),
            pl.BlockSpec((k, n), lambda i: (0, 0)),
        ],
        out_specs=pl.BlockSpec((bm, n), lambda i: (i, 0)),
        out_shape=jax.ShapeDtypeStruct((m, n), _f32),
    )(x, w)


def _fin_body(a0_ref, a1_ref, d0_ref, d1_ref, hs_ref, o_ref):
    deg = jnp.maximum(d0_ref[...][:, 0:1] + d1_ref[...][:, 0:1], 1.0)
    agg = jnp.concatenate([a0_ref[...], a1_ref[...]], axis=1) / deg
    o_ref[...] = jnp.maximum(agg + hs_ref[...], 0.0)


def _finalize(a0, a1, d0, d1, hs, bm):
    m = a0.shape[0]
    return pl.pallas_call(
        _fin_body,
        grid=(m // bm,),
        in_specs=[
            pl.BlockSpec((bm, _HALF), lambda i: (i, 0)),
            pl.BlockSpec((bm, _HALF), lambda i: (i, 0)),
            pl.BlockSpec((bm, 16), lambda i: (i, 0)),
            pl.BlockSpec((bm, 16), lambda i: (i, 0)),
            pl.BlockSpec((bm, _HIDDEN), lambda i: (i, 0)),
        ],
        out_specs=pl.BlockSpec((bm, _HIDDEN), lambda i: (i, 0)),
        out_shape=jax.ShapeDtypeStruct((m, _HIDDEN), _f32),
    )(a0, a1, d0, d1, hs)


def _gru_body(x_ref, h_ref, wih_ref, whh_ref, bih_ref, bhh_ref, o_ref):
    x = x_ref[...]
    h = h_ref[...]
    gi = jnp.dot(x, wih_ref[...], preferred_element_type=_f32) + bih_ref[...]
    gh = jnp.dot(h, whh_ref[...], preferred_element_type=_f32) + bhh_ref[...]
    hd = _HIDDEN
    r = jax.nn.sigmoid(gi[:, :hd] + gh[:, :hd])
    z = jax.nn.sigmoid(gi[:, hd:2 * hd] + gh[:, hd:2 * hd])
    n = jnp.tanh(gi[:, 2 * hd:] + r * gh[:, 2 * hd:])
    hn = (1.0 - z) * n + z * h
    norm = jnp.sqrt(jnp.sum(hn * hn, axis=1, keepdims=True))
    o_ref[...] = hn / jnp.maximum(norm, 1e-12)


def _gru(x, h, wih_t, whh_t, bih, bhh):
    m = x.shape[0]
    return pl.pallas_call(
        _gru_body,
        in_specs=[pl.BlockSpec(a.shape, lambda: (0,) * a.ndim)
                  for a in (x, h, wih_t, whh_t, bih, bhh)],
        out_specs=pl.BlockSpec((m, _HIDDEN), lambda: (0, 0)),
        out_shape=jax.ShapeDtypeStruct((m, _HIDDEN), _f32),
    )(x, h, wih_t, whh_t, bih, bhh)


def _dec_body(e1_ref, e2_ref, w_ref, b_ref, fcw_ref, fcb_ref, o_ref, *, bm):
    e1 = e1_ref[...]
    e2 = e2_ref[...]
    z = jnp.zeros((bm, 1), _f32)
    shifts = []
    for e in (e1, e2):
        shifts.append(jnp.concatenate([z, e[:, :-1]], axis=1))
        shifts.append(e)
        shifts.append(jnp.concatenate([e[:, 1:], z], axis=1))
    acc = jnp.zeros((bm, _HIDDEN), _f32)
    for ch in range(50):
        conv = b_ref[ch]
        for j in range(6):
            conv = conv + shifts[j] * w_ref[6 * ch + j]
        conv = jnp.maximum(conv, 0.0)
        acc = acc + jnp.dot(conv, fcw_ref[ch], preferred_element_type=_f32)
    o_ref[...] = jnp.maximum(acc + fcb_ref[...], 0.0)


def _decoder(e1, e2, conv_w, conv_b, fc_w, fc_b, bm=256):
    m = e1.shape[0]
    wflat = conv_w.reshape(300)
    fcw3 = fc_w.reshape(50, _HIDDEN, _HIDDEN)
    fcb = fc_b.reshape(1, _HIDDEN)
    return pl.pallas_call(
        functools.partial(_dec_body, bm=bm),
        grid=(m // bm,),
        in_specs=[
            pl.BlockSpec((bm, _HIDDEN), lambda i: (i, 0)),
            pl.BlockSpec((bm, _HIDDEN), lambda i: (i, 0)),
            pl.BlockSpec(memory_space=pltpu.SMEM),
            pl.BlockSpec(memory_space=pltpu.SMEM),
            pl.BlockSpec((50, _HIDDEN, _HIDDEN), lambda i: (0, 0, 0)),
            pl.BlockSpec((1, _HIDDEN), lambda i: (0, 0)),
        ],
        out_specs=pl.BlockSpec((bm, _HIDDEN), lambda i: (i, 0)),
        out_shape=jax.ShapeDtypeStruct((m, _HIDDEN), _f32),
    )(e1, e2, wflat, conv_b, fcw3, fcb)


def _score_body(y_ref, t_ref, o_ref):
    o_ref[...] = lax.dot_general(
        y_ref[...], t_ref[...], (((1,), (1,)), ((), ())),
        preferred_element_type=_f32)


def _score(y, table, bn):
    m = y.shape[0]
    n = table.shape[0]
    return pl.pallas_call(
        _score_body,
        grid=(n // bn,),
        in_specs=[
            pl.BlockSpec((m, _HIDDEN), lambda i: (0, 0)),
            pl.BlockSpec((bn, _HIDDEN), lambda i: (i, 0)),
        ],
        out_specs=pl.BlockSpec((m, bn), lambda i: (0, i)),
        out_shape=jax.ShapeDtypeStruct((m, n), _f32),
    )(y, table)


def _pad_to(x, n, val):
    return jnp.concatenate(
        [x, jnp.full((n - x.shape[0],), val, dtype=x.dtype)])


@jax.jit
def kernel(ent_embeds, rel_embeds, time_embeds, rgcn_w_msg, rgcn_w_self,
           gru_w_ih, gru_w_hh, gru_b_ih, gru_b_hh, conve_w, conve_b,
           conve_fc_w, conve_fc_b, convr_w, convr_b, convr_fc_w, convr_fc_b,
           edge_src, edge_dst, edge_type, subj, rel, obj):
    hist = time_embeds.shape[0]
    nlayers = rgcn_w_msg.shape[0]

    src = edge_src.astype(_i32)
    dst = edge_dst.astype(_i32)
    ety = edge_type.astype(_i32)

    gidx_ts, sdst_ts = [], []
    for t in range(hist):
        sh3 = (_NS, _CH_T // 2, _CHUNK)
        ga = _pad_to(src[t], _PART, _NUM_ENTS).reshape(sh3)
        gb = _pad_to(ety[t], _PART, _NUM_RELS).reshape(sh3)
        da = _pad_to(dst[t], _PART, _TRASH).reshape(sh3)
        gidx_ts.append(jnp.concatenate([ga, gb], axis=1))
        sdst_ts.append(jnp.concatenate([da, da], axis=1))

    z64 = jnp.zeros((128, _HALF), _f32)
    z16 = jnp.zeros((128, 16), _f32)
    ones16 = jnp.ones((_CHUNK, 16), _f32)
    zrow = jnp.zeros((1, _HIDDEN), _f32)

    wih_t = gru_w_ih.T
    whh_t = gru_w_hh.T
    bih = gru_b_ih.reshape(1, -1)
    bhh = gru_b_hh.reshape(1, -1)
    relpad = jnp.zeros((232 - _NUM_RELS, _HIDDEN), _f32)
    rel_p = jnp.concatenate([rel_embeds, relpad])

    ent = ent_embeds
    relh_p = rel_p
    for t in range(hist):
        for l in range(nlayers):
            w2 = jnp.concatenate([rgcn_w_msg[l], rgcn_w_self[l]], axis=1)
            hw_hs = _mm(ent, w2, bm=2000)
            rw = _mm(relh_p, rgcn_w_msg[l], bm=232)[:_NUM_RELS]
            table = jnp.concatenate([hw_hs[:, :_HIDDEN], zrow])
            table_h = table.reshape(_NUM_ENTS + 1, _NC, _HALF).transpose(1, 0, 2)
            rw_h = jnp.concatenate([rw, zrow]).reshape(
                _NUM_RELS + 1, _NC, _HALF).transpose(1, 0, 2)
            acc, degp = _get_sc_rgcn()(table_h, rw_h, gidx_ts[t], sdst_ts[t],
                                       z64, z16, ones16)
            ent = _finalize(acc[0, :_NUM_ENTS], acc[1, :_NUM_ENTS],
                            degp[0, :_NUM_ENTS], degp[1, :_NUM_ENTS],
                            hw_hs[:, _HIDDEN:], bm=2000)
        relh_p = _gru(rel_p, relh_p, wih_t, whh_t, bih, bhh)

    relh = relh_p[:_NUM_RELS]

    table2 = jnp.concatenate([ent, relh])
    idx = jnp.concatenate([subj.astype(_i32), obj.astype(_i32),
                           rel.astype(_i32) + _NUM_ENTS]).reshape(_NW, _GPW)
    rows = _get_sc_gather()(table2, idx)
    e_subj = rows[:_BATCH]
    e_obj = rows[_BATCH:2 * _BATCH]
    e_rel = rows[2 * _BATCH:]

    y1 = _decoder(e_subj, e_rel, conve_w, conve_b, conve_fc_w, conve_fc_b)
    y2 = _decoder(e_subj, e_obj, convr_w, convr_b, convr_fc_w, convr_fc_b)

    ent10240 = jnp.concatenate([ent, jnp.zeros((240, _HIDDEN), _f32)])
    ent_logit = _score(y1, ent10240, bn=2048)[:, :_NUM_ENTS]
    relh256 = jnp.concatenate([relh, jnp.zeros((26, _HIDDEN), _f32)])
    rel_logit = _score(y2, relh256, bn=256)[:, :_NUM_RELS]
    return ent_logit, rel_logit


# speedup vs baseline: 1.3370x; 1.1271x over previous
import functools
import jax
import jax.numpy as jnp
from jax import lax
from jax.experimental import pallas as pl
from jax.experimental.pallas import tpu as pltpu
from jax.experimental.pallas import tpu_sc as plsc

_NUM_ENTS = 10000
_NUM_RELS = 230
_HIDDEN = 128
_NUM_EDGES = 160000
_BATCH = 1024

_NC = 2
_NS = 16
_NW = _NC * _NS
_CHUNK = 128
_HALF = _HIDDEN // _NC
_CH_T = 160
_PART = _NS * (_CH_T // 2) * _CHUNK
_ZROW = _NUM_ENTS + _NUM_RELS
_ACC_ROWS = 10112
_RPS = _ACC_ROWS // _NS
_TRASH = _NUM_ENTS

_f32 = jnp.float32
_i32 = jnp.int32


def _edge_loop(tbl, rw_sh, c, gidx_v, sdst_v, rows0, rows1, ones_v,
               sem0, sem1, acc_sh, deg_sh):
    nb = _CH_T // 2
    pltpu.async_copy(tbl.at[gidx_v.at[0]], rows0, sem0)

    def loop_body(j, carry):
        jb = j + nb

        pltpu.make_async_copy(tbl.at[gidx_v.at[j]], rows0, sem0).wait()
        pltpu.async_copy(rw_sh.at[gidx_v.at[jb]], rows1, sem1)
        pltpu.sync_copy(rows0, acc_sh.at[sdst_v.at[j]], add=True)

        @pl.when(lax.rem(j, 2) == c)
        def _():
            pltpu.sync_copy(ones_v, deg_sh.at[sdst_v.at[j]], add=True)

        pltpu.make_async_copy(rw_sh.at[gidx_v.at[jb]], rows1, sem1).wait()

        @pl.when(j < nb - 1)
        def _():
            pltpu.async_copy(tbl.at[gidx_v.at[j + 1]], rows0, sem0)

        pltpu.sync_copy(rows1, acc_sh.at[sdst_v.at[jb]], add=True)
        return carry

    lax.fori_loop(0, nb, loop_body, 0)


def _sc_rgcn_body(table, rw, gidx, sdst, z64, z16, ones16, acc_out, deg_out,
                  gidx_v, sdst_v, rows0, rows1, zero_v, zero16_v, ones_v,
                  sem0, sem1, acc_sh, deg_sh, rw_sh):
    c = lax.axis_index("c")
    s = lax.axis_index("s")
    tbl = table.at[c]

    pltpu.sync_copy(z64, zero_v)
    pltpu.sync_copy(z16, zero16_v)
    pltpu.sync_copy(ones16, ones_v)
    pltpu.sync_copy(gidx.at[s], gidx_v)
    pltpu.sync_copy(sdst.at[s], sdst_v)

    @pl.when(s == 0)
    def _():
        pltpu.sync_copy(rw.at[c], rw_sh)

    base = s * _RPS
    for off in range(0, _RPS, 128):
        n = min(128, _RPS - off)
        pltpu.sync_copy(zero_v.at[pl.ds(0, n)], acc_sh.at[pl.ds(base + off, n)])
        pltpu.sync_copy(zero16_v.at[pl.ds(0, n)], deg_sh.at[pl.ds(base + off, n)])
    plsc.subcore_barrier()

    _edge_loop(tbl, rw_sh, c, gidx_v, sdst_v, rows0, rows1, ones_v,
               sem0, sem1, acc_sh, deg_sh)
    plsc.subcore_barrier()
    pltpu.sync_copy(acc_sh.at[pl.ds(base, _RPS)], acc_out.at[c, pl.ds(base, _RPS)])
    pltpu.sync_copy(deg_sh.at[pl.ds(base, _RPS)], deg_out.at[c, pl.ds(base, _RPS)])


@functools.cache
def _get_sc_rgcn():
    return pl.kernel(
        _sc_rgcn_body,
        out_type=[
            jax.ShapeDtypeStruct((_NC, _ACC_ROWS, _HALF), _f32),
            jax.ShapeDtypeStruct((_NC, _ACC_ROWS, 16), _f32),
        ],
        mesh=plsc.VectorSubcoreMesh(core_axis_name="c", subcore_axis_name="s"),
        compiler_params=pltpu.CompilerParams(use_tc_tiling_on_sc=False),
        scratch_types=[
            pltpu.VMEM((_CH_T, _CHUNK), _i32),
            pltpu.VMEM((_CH_T, _CHUNK), _i32),
            pltpu.VMEM((_CHUNK, _HALF), _f32),
            pltpu.VMEM((_CHUNK, _HALF), _f32),
            pltpu.VMEM((128, _HALF), _f32),
            pltpu.VMEM((128, 16), _f32),
            pltpu.VMEM((_CHUNK, 16), _f32),
            pltpu.SemaphoreType.DMA,
            pltpu.SemaphoreType.DMA,
            pltpu.VMEM_SHARED((_ACC_ROWS, _HALF), _f32),
            pltpu.VMEM_SHARED((_ACC_ROWS, 16), _f32),
            pltpu.VMEM_SHARED((232, _HALF), _f32),
        ],
    )


_GPW = 3 * _BATCH // _NW


def _sc_gather_body(table, idx, out, idx_v, rows_v, sem):
    c = lax.axis_index("c")
    s = lax.axis_index("s")
    wid = c * _NS + s
    pltpu.sync_copy(idx.at[wid], idx_v)
    pltpu.async_copy(table.at[idx_v], rows_v, sem).wait()
    pltpu.sync_copy(rows_v, out.at[pl.ds(wid * _GPW, _GPW)])


@functools.cache
def _get_sc_gather():
    return pl.kernel(
        _sc_gather_body,
        out_type=jax.ShapeDtypeStruct((3 * _BATCH, _HIDDEN), _f32),
        mesh=plsc.VectorSubcoreMesh(core_axis_name="c", subcore_axis_name="s"),
        scratch_types=[
            pltpu.VMEM((_GPW,), _i32),
            pltpu.VMEM((_GPW, _HIDDEN), _f32),
            pltpu.SemaphoreType.DMA,
        ],
    )


def _mm_body(x_ref, w_ref, o_ref):
    o_ref[...] = jnp.dot(x_ref[...], w_ref[...], preferred_element_type=_f32)


def _mm(x, w, bm):
    m, k = x.shape
    n = w.shape[1]
    return pl.pallas_call(
        _mm_body,
        grid=(m // bm,),
        in_specs=[
            pl.BlockSpec((bm, k), lambda i: (i, 0)),
            pl.BlockSpec((k, n), lambda i: (0, 0)),
        ],
        out_specs=pl.BlockSpec((bm, n), lambda i: (i, 0)),
        out_shape=jax.ShapeDtypeStruct((m, n), _f32),
    )(x, w)


def _split3(y, tbl_ref, hs_ref):
    tbl_ref[0] = y[:, :_HALF]
    tbl_ref[1] = y[:, _HALF:_HIDDEN]
    hs_ref[...] = y[:, _HIDDEN:]


def _mm_split_body(x_ref, w_ref, tbl_ref, hs_ref):
    _split3(jnp.dot(x_ref[...], w_ref[...], preferred_element_type=_f32),
            tbl_ref, hs_ref)


def _mm_split(x, w2, bm):
    m = x.shape[0]
    return pl.pallas_call(
        _mm_split_body,
        grid=(m // bm,),
        in_specs=[
            pl.BlockSpec((bm, _HIDDEN), lambda i: (i, 0)),
            pl.BlockSpec((_HIDDEN, 2 * _HIDDEN), lambda i: (0, 0)),
        ],
        out_specs=[
            pl.BlockSpec((_NC, bm, _HALF), lambda i: (0, i, 0)),
            pl.BlockSpec((bm, _HIDDEN), lambda i: (i, 0)),
        ],
        out_shape=[
            jax.ShapeDtypeStruct((_NC, m, _HALF), _f32),
            jax.ShapeDtypeStruct((m, _HIDDEN), _f32),
        ],
    )(x, w2)


def _fin_mm_body(a0_ref, a1_ref, d0_ref, d1_ref, hs_ref, w_ref,
                 tbl_ref, hs_next_ref):
    deg = jnp.maximum(d0_ref[0][:, 0:1] + d1_ref[0][:, 0:1], 1.0)
    agg = jnp.concatenate([a0_ref[0], a1_ref[0]], axis=1) / deg
    e = jnp.maximum(agg + hs_ref[...], 0.0)
    _split3(jnp.dot(e, w_ref[...], preferred_element_type=_f32),
            tbl_ref, hs_next_ref)


def _fin_mm(acc, degp, hs, w2, bm):
    m = hs.shape[0]
    return pl.pallas_call(
        _fin_mm_body,
        grid=(m // bm,),
        in_specs=[
            pl.BlockSpec((1, bm, _HALF), lambda i: (0, i, 0)),
            pl.BlockSpec((1, bm, _HALF), lambda i: (1, i, 0)),
            pl.BlockSpec((1, bm, 16), lambda i: (0, i, 0)),
            pl.BlockSpec((1, bm, 16), lambda i: (1, i, 0)),
            pl.BlockSpec((bm, _HIDDEN), lambda i: (i, 0)),
            pl.BlockSpec((_HIDDEN, 2 * _HIDDEN), lambda i: (0, 0)),
        ],
        out_specs=[
            pl.BlockSpec((_NC, bm, _HALF), lambda i: (0, i, 0)),
            pl.BlockSpec((bm, _HIDDEN), lambda i: (i, 0)),
        ],
        out_shape=[
            jax.ShapeDtypeStruct((_NC, m, _HALF), _f32),
            jax.ShapeDtypeStruct((m, _HIDDEN), _f32),
        ],
    )(acc, acc, degp, degp, hs, w2)


def _fin_body(a0_ref, a1_ref, d0_ref, d1_ref, hs_ref, o_ref):
    deg = jnp.maximum(d0_ref[...][:, 0:1] + d1_ref[...][:, 0:1], 1.0)
    agg = jnp.concatenate([a0_ref[...], a1_ref[...]], axis=1) / deg
    o_ref[...] = jnp.maximum(agg + hs_ref[...], 0.0)


def _finalize(a0, a1, d0, d1, hs, bm):
    m = a0.shape[0]
    return pl.pallas_call(
        _fin_body,
        grid=(m // bm,),
        in_specs=[
            pl.BlockSpec((bm, _HALF), lambda i: (i, 0)),
            pl.BlockSpec((bm, _HALF), lambda i: (i, 0)),
            pl.BlockSpec((bm, 16), lambda i: (i, 0)),
            pl.BlockSpec((bm, 16), lambda i: (i, 0)),
            pl.BlockSpec((bm, _HIDDEN), lambda i: (i, 0)),
        ],
        out_specs=pl.BlockSpec((bm, _HIDDEN), lambda i: (i, 0)),
        out_shape=jax.ShapeDtypeStruct((m, _HIDDEN), _f32),
    )(a0, a1, d0, d1, hs)


def _gru_body(x_ref, h_ref, wih_ref, whh_ref, bih_ref, bhh_ref, wm_ref,
              o_ref, rw0_ref, rw1_ref):
    x = x_ref[...]
    h = h_ref[...]
    gi = jnp.dot(x, wih_ref[...], preferred_element_type=_f32) + bih_ref[...]
    gh = jnp.dot(h, whh_ref[...], preferred_element_type=_f32) + bhh_ref[...]
    hd = _HIDDEN
    r = jax.nn.sigmoid(gi[:, :hd] + gh[:, :hd])
    z = jax.nn.sigmoid(gi[:, hd:2 * hd] + gh[:, hd:2 * hd])
    n = jnp.tanh(gi[:, 2 * hd:] + r * gh[:, 2 * hd:])
    hn = (1.0 - z) * n + z * h
    norm = jnp.sqrt(jnp.sum(hn * hn, axis=1, keepdims=True))
    relh = hn / jnp.maximum(norm, 1e-12)
    o_ref[...] = relh
    rwc = jnp.dot(relh, wm_ref[...], preferred_element_type=_f32)
    rw0_ref[0] = rwc[:, :_HALF]
    rw0_ref[1] = rwc[:, _HALF:_HIDDEN]
    rw1_ref[0] = rwc[:, _HIDDEN:_HIDDEN + _HALF]
    rw1_ref[1] = rwc[:, _HIDDEN + _HALF:]


def _gru(x, h, wih_t, whh_t, bih, bhh, wm01):
    m = x.shape[0]
    args = (x, h, wih_t, whh_t, bih, bhh, wm01)
    return pl.pallas_call(
        _gru_body,
        in_specs=[pl.BlockSpec(a.shape, lambda: (0,) * a.ndim) for a in args],
        out_specs=[
            pl.BlockSpec((m, _HIDDEN), lambda: (0, 0)),
            pl.BlockSpec((_NC, m, _HALF), lambda: (0, 0, 0)),
            pl.BlockSpec((_NC, m, _HALF), lambda: (0, 0, 0)),
        ],
        out_shape=[
            jax.ShapeDtypeStruct((m, _HIDDEN), _f32),
            jax.ShapeDtypeStruct((_NC, m, _HALF), _f32),
            jax.ShapeDtypeStruct((_NC, m, _HALF), _f32),
        ],
    )(*args)


def _dec_body(e1_ref, e2_ref, w_ref, b_ref, fcw_ref, fcb_ref, o_ref, *, bm):
    e1 = e1_ref[...]
    e2 = e2_ref[...]
    z = jnp.zeros((bm, 1), _f32)
    shifts = []
    for e in (e1, e2):
        shifts.append(jnp.concatenate([z, e[:, :-1]], axis=1))
        shifts.append(e)
        shifts.append(jnp.concatenate([e[:, 1:], z], axis=1))
    acc = jnp.zeros((bm, _HIDDEN), _f32)
    for ch in range(50):
        conv = b_ref[ch]
        for j in range(6):
            conv = conv + shifts[j] * w_ref[6 * ch + j]
        conv = jnp.maximum(conv, 0.0)
        acc = acc + jnp.dot(conv, fcw_ref[ch], preferred_element_type=_f32)
    o_ref[...] = jnp.maximum(acc + fcb_ref[...], 0.0)


def _decoder(e1, e2, conv_w, conv_b, fc_w, fc_b, bm=256):
    m = e1.shape[0]
    wflat = conv_w.reshape(300)
    fcw3 = fc_w.reshape(50, _HIDDEN, _HIDDEN)
    fcb = fc_b.reshape(1, _HIDDEN)
    return pl.pallas_call(
        functools.partial(_dec_body, bm=bm),
        grid=(m // bm,),
        in_specs=[
            pl.BlockSpec((bm, _HIDDEN), lambda i: (i, 0)),
            pl.BlockSpec((bm, _HIDDEN), lambda i: (i, 0)),
            pl.BlockSpec(memory_space=pltpu.SMEM),
            pl.BlockSpec(memory_space=pltpu.SMEM),
            pl.BlockSpec((50, _HIDDEN, _HIDDEN), lambda i: (0, 0, 0)),
            pl.BlockSpec((1, _HIDDEN), lambda i: (0, 0)),
        ],
        out_specs=pl.BlockSpec((bm, _HIDDEN), lambda i: (i, 0)),
        out_shape=jax.ShapeDtypeStruct((m, _HIDDEN), _f32),
    )(e1, e2, wflat, conv_b, fcw3, fcb)


def _score_body(y_ref, t_ref, o_ref):
    o_ref[...] = lax.dot_general(
        y_ref[...], t_ref[...], (((1,), (1,)), ((), ())),
        preferred_element_type=_f32)


def _score(y, table, bn):
    m = y.shape[0]
    n = table.shape[0]
    return pl.pallas_call(
        _score_body,
        grid=(n // bn,),
        in_specs=[
            pl.BlockSpec((m, _HIDDEN), lambda i: (0, 0)),
            pl.BlockSpec((bn, _HIDDEN), lambda i: (i, 0)),
        ],
        out_specs=pl.BlockSpec((m, bn), lambda i: (0, i)),
        out_shape=jax.ShapeDtypeStruct((m, n), _f32),
    )(y, table)


def _pad_to(x, n, val):
    return jnp.concatenate(
        [x, jnp.full((n - x.shape[0],), val, dtype=x.dtype)])


@jax.jit
def kernel(ent_embeds, rel_embeds, time_embeds, rgcn_w_msg, rgcn_w_self,
           gru_w_ih, gru_w_hh, gru_b_ih, gru_b_hh, conve_w, conve_b,
           conve_fc_w, conve_fc_b, convr_w, convr_b, convr_fc_w, convr_fc_b,
           edge_src, edge_dst, edge_type, subj, rel, obj):
    hist = time_embeds.shape[0]
    nlayers = rgcn_w_msg.shape[0]

    src = edge_src.astype(_i32)
    dst = edge_dst.astype(_i32)
    ety = edge_type.astype(_i32)

    gidx_ts, sdst_ts = [], []
    for t in range(hist):
        sh3 = (_NS, _CH_T // 2, _CHUNK)
        ga = _pad_to(src[t], _PART, 0).reshape(sh3)
        gb = _pad_to(ety[t], _PART, 0).reshape(sh3)
        da = _pad_to(dst[t], _PART, _TRASH).reshape(sh3)
        gidx_ts.append(jnp.concatenate([ga, gb], axis=1))
        sdst_ts.append(jnp.concatenate([da, da], axis=1))

    z64 = jnp.zeros((128, _HALF), _f32)
    z16 = jnp.zeros((128, 16), _f32)
    ones16 = jnp.ones((_CHUNK, 16), _f32)

    wih_t = gru_w_ih.T
    whh_t = gru_w_hh.T
    bih = gru_b_ih.reshape(1, -1)
    bhh = gru_b_hh.reshape(1, -1)
    relpad = jnp.zeros((232 - _NUM_RELS, _HIDDEN), _f32)
    rel_p = jnp.concatenate([rel_embeds, relpad])

    w2s = [jnp.concatenate([rgcn_w_msg[l], rgcn_w_self[l]], axis=1)
           for l in range(nlayers)]
    wm01 = jnp.concatenate([rgcn_w_msg[0], rgcn_w_msg[1]], axis=1)

    tbl, hs = _mm_split(ent_embeds, w2s[0], bm=2000)
    rwc0 = _mm(rel_p, wm01, bm=232)
    rw0 = rwc0[:, :_HIDDEN].reshape(232, _NC, _HALF).transpose(1, 0, 2)
    rw1 = rwc0[:, _HIDDEN:].reshape(232, _NC, _HALF).transpose(1, 0, 2)

    relh_p = rel_p
    for t in range(hist):
        rw_t = (rw0, rw1)
        for l in range(nlayers):
            acc, degp = _get_sc_rgcn()(tbl, rw_t[l], gidx_ts[t], sdst_ts[t],
                                       z64, z16, ones16)
            if t == hist - 1 and l == nlayers - 1:
                ent = _finalize(acc[0, :_NUM_ENTS], acc[1, :_NUM_ENTS],
                                degp[0, :_NUM_ENTS], degp[1, :_NUM_ENTS],
                                hs, bm=2000)
            else:
                tbl, hs = _fin_mm(acc, degp, hs, w2s[(l + 1) % nlayers],
                                  bm=2000)
        relh_p, rw0, rw1 = _gru(rel_p, relh_p, wih_t, whh_t, bih, bhh, wm01)

    relh = relh_p[:_NUM_RELS]

    table2 = jnp.concatenate([ent, relh])
    idx = jnp.concatenate([subj.astype(_i32), obj.astype(_i32),
                           rel.astype(_i32) + _NUM_ENTS]).reshape(_NW, _GPW)
    rows = _get_sc_gather()(table2, idx)
    e_subj = rows[:_BATCH]
    e_obj = rows[_BATCH:2 * _BATCH]
    e_rel = rows[2 * _BATCH:]

    y1 = _decoder(e_subj, e_rel, conve_w, conve_b, conve_fc_w, conve_fc_b)
    y2 = _decoder(e_subj, e_obj, convr_w, convr_b, convr_fc_w, convr_fc_b)

    ent10240 = jnp.concatenate([ent, jnp.zeros((240, _HIDDEN), _f32)])
    ent_logit = _score(y1, ent10240, bn=2048)[:, :_NUM_ENTS]
    relh256 = jnp.concatenate([relh, jnp.zeros((26, _HIDDEN), _f32)])
    rel_logit = _score(y2, relh256, bn=256)[:, :_NUM_RELS]
    return ent_logit, rel_logit


# speedup vs baseline: 1.4040x; 1.0502x over previous
import functools
import jax
import jax.numpy as jnp
from jax import lax
from jax.experimental import pallas as pl
from jax.experimental.pallas import tpu as pltpu
from jax.experimental.pallas import tpu_sc as plsc

_NUM_ENTS = 10000
_NUM_RELS = 230
_HIDDEN = 128
_NUM_EDGES = 160000
_BATCH = 1024

_NC = 2
_NS = 16
_NW = _NC * _NS
_CHUNK = 128
_HALF = _HIDDEN // _NC
_CH_T = 160
_PART = _NS * (_CH_T // 2) * _CHUNK
_ZROW = _NUM_ENTS + _NUM_RELS
_ACC_ROWS = 10112
_RPS = _ACC_ROWS // _NS
_TRASH = _NUM_ENTS

_f32 = jnp.float32
_i32 = jnp.int32


def _edge_loop(tbl, rw_sh, c, gidx_v, sdst_v, rows0, rows1, rows2, ones_v,
               sem0, sem1, sem2, acc_sh, deg_sh):
    nb = _CH_T // 2
    pltpu.async_copy(tbl.at[gidx_v.at[0]], rows0, sem0)
    pltpu.async_copy(tbl.at[gidx_v.at[1]], rows2, sem2)

    def half_body(j, ra, sa, carry):
        jb = j + nb
        pltpu.make_async_copy(tbl.at[gidx_v.at[j]], ra, sa).wait()
        pltpu.async_copy(rw_sh.at[gidx_v.at[jb]], rows1, sem1)
        pltpu.sync_copy(ra, acc_sh.at[sdst_v.at[j]], add=True)

        @pl.when(lax.rem(j, 2) == c)
        def _():
            pltpu.sync_copy(ones_v, deg_sh.at[sdst_v.at[j]], add=True)

        @pl.when(j < nb - 2)
        def _():
            pltpu.async_copy(tbl.at[gidx_v.at[j + 2]], ra, sa)

        pltpu.make_async_copy(rw_sh.at[gidx_v.at[jb]], rows1, sem1).wait()
        pltpu.sync_copy(rows1, acc_sh.at[sdst_v.at[jb]], add=True)
        return carry

    def loop_body(i, carry):
        carry = half_body(2 * i, rows0, sem0, carry)
        carry = half_body(2 * i + 1, rows2, sem2, carry)
        return carry

    lax.fori_loop(0, nb // 2, loop_body, 0)


def _sc_rgcn_body(table, rw, gidx, sdst, z64, z16, ones16, acc_out, deg_out,
                  gidx_v, sdst_v, rows0, rows1, rows2, zero_v, zero16_v,
                  ones_v, sem0, sem1, sem2, acc_sh, deg_sh, rw_sh):
    c = lax.axis_index("c")
    s = lax.axis_index("s")
    tbl = table.at[c]

    pltpu.sync_copy(z64, zero_v)
    pltpu.sync_copy(z16, zero16_v)
    pltpu.sync_copy(ones16, ones_v)
    pltpu.sync_copy(gidx.at[s], gidx_v)
    pltpu.sync_copy(sdst.at[s], sdst_v)

    @pl.when(s == 0)
    def _():
        pltpu.sync_copy(rw.at[c], rw_sh)

    base = s * _RPS
    for off in range(0, _RPS, 128):
        n = min(128, _RPS - off)
        pltpu.sync_copy(zero_v.at[pl.ds(0, n)], acc_sh.at[pl.ds(base + off, n)])
        pltpu.sync_copy(zero16_v.at[pl.ds(0, n)], deg_sh.at[pl.ds(base + off, n)])
    plsc.subcore_barrier()

    _edge_loop(tbl, rw_sh, c, gidx_v, sdst_v, rows0, rows1, rows2, ones_v,
               sem0, sem1, sem2, acc_sh, deg_sh)
    plsc.subcore_barrier()
    pltpu.sync_copy(acc_sh.at[pl.ds(base, _RPS)], acc_out.at[c, pl.ds(base, _RPS)])
    pltpu.sync_copy(deg_sh.at[pl.ds(base, _RPS)], deg_out.at[c, pl.ds(base, _RPS)])


@functools.cache
def _get_sc_rgcn():
    return pl.kernel(
        _sc_rgcn_body,
        out_type=[
            jax.ShapeDtypeStruct((_NC, _ACC_ROWS, _HALF), _f32),
            jax.ShapeDtypeStruct((_NC, _ACC_ROWS, 16), _f32),
        ],
        mesh=plsc.VectorSubcoreMesh(core_axis_name="c", subcore_axis_name="s"),
        compiler_params=pltpu.CompilerParams(use_tc_tiling_on_sc=False),
        scratch_types=[
            pltpu.VMEM((_CH_T, _CHUNK), _i32),
            pltpu.VMEM((_CH_T, _CHUNK), _i32),
            pltpu.VMEM((_CHUNK, _HALF), _f32),
            pltpu.VMEM((_CHUNK, _HALF), _f32),
            pltpu.VMEM((_CHUNK, _HALF), _f32),
            pltpu.VMEM((128, _HALF), _f32),
            pltpu.VMEM((128, 16), _f32),
            pltpu.VMEM((_CHUNK, 16), _f32),
            pltpu.SemaphoreType.DMA,
            pltpu.SemaphoreType.DMA,
            pltpu.SemaphoreType.DMA,
            pltpu.VMEM_SHARED((_ACC_ROWS, _HALF), _f32),
            pltpu.VMEM_SHARED((_ACC_ROWS, 16), _f32),
            pltpu.VMEM_SHARED((232, _HALF), _f32),
        ],
    )


_GPW = 3 * _BATCH // _NW


def _sc_gather_body(table, idx, out, idx_v, rows_v, sem):
    c = lax.axis_index("c")
    s = lax.axis_index("s")
    wid = c * _NS + s
    pltpu.sync_copy(idx.at[wid], idx_v)
    pltpu.async_copy(table.at[idx_v], rows_v, sem).wait()
    pltpu.sync_copy(rows_v, out.at[pl.ds(wid * _GPW, _GPW)])


@functools.cache
def _get_sc_gather():
    return pl.kernel(
        _sc_gather_body,
        out_type=jax.ShapeDtypeStruct((3 * _BATCH, _HIDDEN), _f32),
        mesh=plsc.VectorSubcoreMesh(core_axis_name="c", subcore_axis_name="s"),
        scratch_types=[
            pltpu.VMEM((_GPW,), _i32),
            pltpu.VMEM((_GPW, _HIDDEN), _f32),
            pltpu.SemaphoreType.DMA,
        ],
    )


def _mm_body(x_ref, w_ref, o_ref):
    o_ref[...] = jnp.dot(x_ref[...], w_ref[...], preferred_element_type=_f32)


def _mm(x, w, bm):
    m, k = x.shape
    n = w.shape[1]
    return pl.pallas_call(
        _mm_body,
        grid=(m // bm,),
        in_specs=[
            pl.BlockSpec((bm, k), lambda i: (i, 0)),
            pl.BlockSpec((k, n), lambda i: (0, 0)),
        ],
        out_specs=pl.BlockSpec((bm, n), lambda i: (i, 0)),
        out_shape=jax.ShapeDtypeStruct((m, n), _f32),
    )(x, w)


def _split3(y, tbl_ref, hs_ref):
    tbl_ref[0] = y[:, :_HALF]
    tbl_ref[1] = y[:, _HALF:_HIDDEN]
    hs_ref[...] = y[:, _HIDDEN:]


def _mm_split_body(x_ref, w_ref, tbl_ref, hs_ref):
    _split3(jnp.dot(x_ref[...], w_ref[...], preferred_element_type=_f32),
            tbl_ref, hs_ref)


def _mm_split(x, w2, bm):
    m = x.shape[0]
    return pl.pallas_call(
        _mm_split_body,
        grid=(m // bm,),
        in_specs=[
            pl.BlockSpec((bm, _HIDDEN), lambda i: (i, 0)),
            pl.BlockSpec((_HIDDEN, 2 * _HIDDEN), lambda i: (0, 0)),
        ],
        out_specs=[
            pl.BlockSpec((_NC, bm, _HALF), lambda i: (0, i, 0)),
            pl.BlockSpec((bm, _HIDDEN), lambda i: (i, 0)),
        ],
        out_shape=[
            jax.ShapeDtypeStruct((_NC, m, _HALF), _f32),
            jax.ShapeDtypeStruct((m, _HIDDEN), _f32),
        ],
    )(x, w2)


def _fin_mm_body(a0_ref, a1_ref, d0_ref, d1_ref, hs_ref, w_ref,
                 tbl_ref, hs_next_ref):
    deg = jnp.maximum(d0_ref[0][:, 0:1] + d1_ref[0][:, 0:1], 1.0)
    agg = jnp.concatenate([a0_ref[0], a1_ref[0]], axis=1) / deg
    e = jnp.maximum(agg + hs_ref[...], 0.0)
    _split3(jnp.dot(e, w_ref[...], preferred_element_type=_f32),
            tbl_ref, hs_next_ref)


def _fin_mm(acc, degp, hs, w2, bm):
    m = hs.shape[0]
    return pl.pallas_call(
        _fin_mm_body,
        grid=(m // bm,),
        in_specs=[
            pl.BlockSpec((1, bm, _HALF), lambda i: (0, i, 0)),
            pl.BlockSpec((1, bm, _HALF), lambda i: (1, i, 0)),
            pl.BlockSpec((1, bm, 16), lambda i: (0, i, 0)),
            pl.BlockSpec((1, bm, 16), lambda i: (1, i, 0)),
            pl.BlockSpec((bm, _HIDDEN), lambda i: (i, 0)),
            pl.BlockSpec((_HIDDEN, 2 * _HIDDEN), lambda i: (0, 0)),
        ],
        out_specs=[
            pl.BlockSpec((_NC, bm, _HALF), lambda i: (0, i, 0)),
            pl.BlockSpec((bm, _HIDDEN), lambda i: (i, 0)),
        ],
        out_shape=[
            jax.ShapeDtypeStruct((_NC, m, _HALF), _f32),
            jax.ShapeDtypeStruct((m, _HIDDEN), _f32),
        ],
    )(acc, acc, degp, degp, hs, w2)


def _fin_body(a0_ref, a1_ref, d0_ref, d1_ref, hs_ref, o_ref):
    deg = jnp.maximum(d0_ref[...][:, 0:1] + d1_ref[...][:, 0:1], 1.0)
    agg = jnp.concatenate([a0_ref[...], a1_ref[...]], axis=1) / deg
    o_ref[...] = jnp.maximum(agg + hs_ref[...], 0.0)


def _finalize(a0, a1, d0, d1, hs, bm):
    m = a0.shape[0]
    return pl.pallas_call(
        _fin_body,
        grid=(m // bm,),
        in_specs=[
            pl.BlockSpec((bm, _HALF), lambda i: (i, 0)),
            pl.BlockSpec((bm, _HALF), lambda i: (i, 0)),
            pl.BlockSpec((bm, 16), lambda i: (i, 0)),
            pl.BlockSpec((bm, 16), lambda i: (i, 0)),
            pl.BlockSpec((bm, _HIDDEN), lambda i: (i, 0)),
        ],
        out_specs=pl.BlockSpec((bm, _HIDDEN), lambda i: (i, 0)),
        out_shape=jax.ShapeDtypeStruct((m, _HIDDEN), _f32),
    )(a0, a1, d0, d1, hs)


def _gru_body(x_ref, h_ref, wih_ref, whh_ref, bih_ref, bhh_ref, wm_ref,
              o_ref, rw0_ref, rw1_ref):
    x = x_ref[...]
    h = h_ref[...]
    gi = jnp.dot(x, wih_ref[...], preferred_element_type=_f32) + bih_ref[...]
    gh = jnp.dot(h, whh_ref[...], preferred_element_type=_f32) + bhh_ref[...]
    hd = _HIDDEN
    r = jax.nn.sigmoid(gi[:, :hd] + gh[:, :hd])
    z = jax.nn.sigmoid(gi[:, hd:2 * hd] + gh[:, hd:2 * hd])
    n = jnp.tanh(gi[:, 2 * hd:] + r * gh[:, 2 * hd:])
    hn = (1.0 - z) * n + z * h
    norm = jnp.sqrt(jnp.sum(hn * hn, axis=1, keepdims=True))
    relh = hn / jnp.maximum(norm, 1e-12)
    o_ref[...] = relh
    rwc = jnp.dot(relh, wm_ref[...], preferred_element_type=_f32)
    rw0_ref[0] = rwc[:, :_HALF]
    rw0_ref[1] = rwc[:, _HALF:_HIDDEN]
    rw1_ref[0] = rwc[:, _HIDDEN:_HIDDEN + _HALF]
    rw1_ref[1] = rwc[:, _HIDDEN + _HALF:]


def _gru(x, h, wih_t, whh_t, bih, bhh, wm01):
    m = x.shape[0]
    args = (x, h, wih_t, whh_t, bih, bhh, wm01)
    return pl.pallas_call(
        _gru_body,
        in_specs=[pl.BlockSpec(a.shape, lambda: (0,) * a.ndim) for a in args],
        out_specs=[
            pl.BlockSpec((m, _HIDDEN), lambda: (0, 0)),
            pl.BlockSpec((_NC, m, _HALF), lambda: (0, 0, 0)),
            pl.BlockSpec((_NC, m, _HALF), lambda: (0, 0, 0)),
        ],
        out_shape=[
            jax.ShapeDtypeStruct((m, _HIDDEN), _f32),
            jax.ShapeDtypeStruct((_NC, m, _HALF), _f32),
            jax.ShapeDtypeStruct((_NC, m, _HALF), _f32),
        ],
    )(*args)


def _dec_body(e1_ref, e2_ref, w_ref, b_ref, fcw_ref, fcb_ref, o_ref, *, bm):
    e1 = e1_ref[...]
    e2 = e2_ref[...]
    z = jnp.zeros((bm, 1), _f32)
    shifts = []
    for e in (e1, e2):
        shifts.append(jnp.concatenate([z, e[:, :-1]], axis=1))
        shifts.append(e)
        shifts.append(jnp.concatenate([e[:, 1:], z], axis=1))
    acc = jnp.zeros((bm, _HIDDEN), _f32)
    for ch in range(50):
        conv = b_ref[ch]
        for j in range(6):
            conv = conv + shifts[j] * w_ref[6 * ch + j]
        conv = jnp.maximum(conv, 0.0)
        acc = acc + jnp.dot(conv, fcw_ref[ch], preferred_element_type=_f32)
    o_ref[...] = jnp.maximum(acc + fcb_ref[...], 0.0)


def _decoder(e1, e2, conv_w, conv_b, fc_w, fc_b, bm=256):
    m = e1.shape[0]
    wflat = conv_w.reshape(300)
    fcw3 = fc_w.reshape(50, _HIDDEN, _HIDDEN)
    fcb = fc_b.reshape(1, _HIDDEN)
    return pl.pallas_call(
        functools.partial(_dec_body, bm=bm),
        grid=(m // bm,),
        in_specs=[
            pl.BlockSpec((bm, _HIDDEN), lambda i: (i, 0)),
            pl.BlockSpec((bm, _HIDDEN), lambda i: (i, 0)),
            pl.BlockSpec(memory_space=pltpu.SMEM),
            pl.BlockSpec(memory_space=pltpu.SMEM),
            pl.BlockSpec((50, _HIDDEN, _HIDDEN), lambda i: (0, 0, 0)),
            pl.BlockSpec((1, _HIDDEN), lambda i: (0, 0)),
        ],
        out_specs=pl.BlockSpec((bm, _HIDDEN), lambda i: (i, 0)),
        out_shape=jax.ShapeDtypeStruct((m, _HIDDEN), _f32),
    )(e1, e2, wflat, conv_b, fcw3, fcb)


def _score_body(y_ref, t_ref, o_ref):
    o_ref[...] = lax.dot_general(
        y_ref[...], t_ref[...], (((1,), (1,)), ((), ())),
        preferred_element_type=_f32)


def _score(y, table, bn):
    m = y.shape[0]
    n = table.shape[0]
    return pl.pallas_call(
        _score_body,
        grid=(n // bn,),
        in_specs=[
            pl.BlockSpec((m, _HIDDEN), lambda i: (0, 0)),
            pl.BlockSpec((bn, _HIDDEN), lambda i: (i, 0)),
        ],
        out_specs=pl.BlockSpec((m, bn), lambda i: (0, i)),
        out_shape=jax.ShapeDtypeStruct((m, n), _f32),
    )(y, table)


def _pad_to(x, n, val):
    return jnp.concatenate(
        [x, jnp.full((n - x.shape[0],), val, dtype=x.dtype)])


@jax.jit
def kernel(ent_embeds, rel_embeds, time_embeds, rgcn_w_msg, rgcn_w_self,
           gru_w_ih, gru_w_hh, gru_b_ih, gru_b_hh, conve_w, conve_b,
           conve_fc_w, conve_fc_b, convr_w, convr_b, convr_fc_w, convr_fc_b,
           edge_src, edge_dst, edge_type, subj, rel, obj):
    hist = time_embeds.shape[0]
    nlayers = rgcn_w_msg.shape[0]

    src = edge_src.astype(_i32)
    dst = edge_dst.astype(_i32)
    ety = edge_type.astype(_i32)

    gidx_ts, sdst_ts = [], []
    for t in range(hist):
        sh3 = (_NS, _CH_T // 2, _CHUNK)
        ga = _pad_to(src[t], _PART, 0).reshape(sh3)
        gb = _pad_to(ety[t], _PART, 0).reshape(sh3)
        da = _pad_to(dst[t], _PART, _TRASH).reshape(sh3)
        gidx_ts.append(jnp.concatenate([ga, gb], axis=1))
        sdst_ts.append(jnp.concatenate([da, da], axis=1))

    z64 = jnp.zeros((128, _HALF), _f32)
    z16 = jnp.zeros((128, 16), _f32)
    ones16 = jnp.ones((_CHUNK, 16), _f32)

    wih_t = gru_w_ih.T
    whh_t = gru_w_hh.T
    bih = gru_b_ih.reshape(1, -1)
    bhh = gru_b_hh.reshape(1, -1)
    relpad = jnp.zeros((232 - _NUM_RELS, _HIDDEN), _f32)
    rel_p = jnp.concatenate([rel_embeds, relpad])

    w2s = [jnp.concatenate([rgcn_w_msg[l], rgcn_w_self[l]], axis=1)
           for l in range(nlayers)]
    wm01 = jnp.concatenate([rgcn_w_msg[0], rgcn_w_msg[1]], axis=1)

    tbl, hs = _mm_split(ent_embeds, w2s[0], bm=2000)
    rwc0 = _mm(rel_p, wm01, bm=232)
    rw0 = rwc0[:, :_HIDDEN].reshape(232, _NC, _HALF).transpose(1, 0, 2)
    rw1 = rwc0[:, _HIDDEN:].reshape(232, _NC, _HALF).transpose(1, 0, 2)

    relh_p = rel_p
    for t in range(hist):
        rw_t = (rw0, rw1)
        for l in range(nlayers):
            acc, degp = _get_sc_rgcn()(tbl, rw_t[l], gidx_ts[t], sdst_ts[t],
                                       z64, z16, ones16)
            if t == hist - 1 and l == nlayers - 1:
                ent = _finalize(acc[0, :_NUM_ENTS], acc[1, :_NUM_ENTS],
                                degp[0, :_NUM_ENTS], degp[1, :_NUM_ENTS],
                                hs, bm=2000)
            else:
                tbl, hs = _fin_mm(acc, degp, hs, w2s[(l + 1) % nlayers],
                                  bm=2000)
        relh_p, rw0, rw1 = _gru(rel_p, relh_p, wih_t, whh_t, bih, bhh, wm01)

    relh = relh_p[:_NUM_RELS]

    table2 = jnp.concatenate([ent, relh])
    idx = jnp.concatenate([subj.astype(_i32), obj.astype(_i32),
                           rel.astype(_i32) + _NUM_ENTS]).reshape(_NW, _GPW)
    rows = _get_sc_gather()(table2, idx)
    e_subj = rows[:_BATCH]
    e_obj = rows[_BATCH:2 * _BATCH]
    e_rel = rows[2 * _BATCH:]

    y1 = _decoder(e_subj, e_rel, conve_w, conve_b, conve_fc_w, conve_fc_b)
    y2 = _decoder(e_subj, e_obj, convr_w, convr_b, convr_fc_w, convr_fc_b)

    ent10240 = jnp.concatenate([ent, jnp.zeros((240, _HIDDEN), _f32)])
    ent_logit = _score(y1, ent10240, bn=2048)[:, :_NUM_ENTS]
    relh256 = jnp.concatenate([relh, jnp.zeros((26, _HIDDEN), _f32)])
    rel_logit = _score(y2, relh256, bn=256)[:, :_NUM_RELS]
    return ent_logit, rel_logit


# speedup vs baseline: 1.4527x; 1.0346x over previous
import functools
import jax
import jax.numpy as jnp
from jax import lax
from jax.experimental import pallas as pl
from jax.experimental.pallas import tpu as pltpu
from jax.experimental.pallas import tpu_sc as plsc

_NUM_ENTS = 10000
_NUM_RELS = 230
_HIDDEN = 128
_NUM_EDGES = 160000
_BATCH = 1024

_NC = 2
_NS = 16
_NW = _NC * _NS
_CHUNK = 128
_HALF = _HIDDEN // _NC
_CH_T = 160
_PART = _NS * (_CH_T // 2) * _CHUNK
_ZROW = _NUM_ENTS + _NUM_RELS
_ACC_ROWS = 10112
_RPS = _ACC_ROWS // _NS
_TRASH = _NUM_ENTS

_f32 = jnp.float32
_i32 = jnp.int32


def _edge_loop(tbl, rw_sh, c, gidx_v, sdst_v, rows0, rows1, rows2, rows3,
               ones_v, sem0, sem1, sem2, sem3, acc_sh, deg_sh):
    nb = _CH_T // 2
    pltpu.async_copy(tbl.at[gidx_v.at[0]], rows0, sem0)
    pltpu.async_copy(tbl.at[gidx_v.at[1]], rows2, sem2)
    pltpu.async_copy(rw_sh.at[gidx_v.at[nb]], rows1, sem1)
    pltpu.async_copy(rw_sh.at[gidx_v.at[nb + 1]], rows3, sem3)

    def half_body(j, ra, sa, rb, sb, carry):
        jb = j + nb
        pltpu.make_async_copy(tbl.at[gidx_v.at[j]], ra, sa).wait()
        pltpu.sync_copy(ra, acc_sh.at[sdst_v.at[j]], add=True)

        @pl.when(lax.rem(j, 2) == c)
        def _():
            pltpu.sync_copy(ones_v, deg_sh.at[sdst_v.at[j]], add=True)

        @pl.when(j < nb - 2)
        def _():
            pltpu.async_copy(tbl.at[gidx_v.at[j + 2]], ra, sa)

        pltpu.make_async_copy(rw_sh.at[gidx_v.at[jb]], rb, sb).wait()
        pltpu.sync_copy(rb, acc_sh.at[sdst_v.at[jb]], add=True)

        @pl.when(j < nb - 2)
        def _():
            pltpu.async_copy(rw_sh.at[gidx_v.at[jb + 2]], rb, sb)

        return carry

    def loop_body(i, carry):
        carry = half_body(2 * i, rows0, sem0, rows1, sem1, carry)
        carry = half_body(2 * i + 1, rows2, sem2, rows3, sem3, carry)
        return carry

    lax.fori_loop(0, nb // 2, loop_body, 0)


def _sc_rgcn_body(table, rw, gidx, sdst, z64, z16, ones16, acc_out, deg_out,
                  gidx_v, sdst_v, rows0, rows1, rows2, rows3, zero_v,
                  zero16_v, ones_v, sem0, sem1, sem2, sem3,
                  acc_sh, deg_sh, rw_sh):
    c = lax.axis_index("c")
    s = lax.axis_index("s")
    tbl = table.at[c]

    pltpu.sync_copy(z64, zero_v)
    pltpu.sync_copy(z16, zero16_v)
    pltpu.sync_copy(ones16, ones_v)
    pltpu.sync_copy(gidx.at[s], gidx_v)
    pltpu.sync_copy(sdst.at[s], sdst_v)

    @pl.when(s == 0)
    def _():
        pltpu.sync_copy(rw.at[c], rw_sh)

    base = s * _RPS
    for off in range(0, _RPS, 128):
        n = min(128, _RPS - off)
        pltpu.sync_copy(zero_v.at[pl.ds(0, n)], acc_sh.at[pl.ds(base + off, n)])
        pltpu.sync_copy(zero16_v.at[pl.ds(0, n)], deg_sh.at[pl.ds(base + off, n)])
    plsc.subcore_barrier()

    _edge_loop(tbl, rw_sh, c, gidx_v, sdst_v, rows0, rows1, rows2, rows3,
               ones_v, sem0, sem1, sem2, sem3, acc_sh, deg_sh)
    plsc.subcore_barrier()
    pltpu.sync_copy(acc_sh.at[pl.ds(base, _RPS)], acc_out.at[c, pl.ds(base, _RPS)])
    pltpu.sync_copy(deg_sh.at[pl.ds(base, _RPS)], deg_out.at[c, pl.ds(base, _RPS)])


@functools.cache
def _get_sc_rgcn():
    return pl.kernel(
        _sc_rgcn_body,
        out_type=[
            jax.ShapeDtypeStruct((_NC, _ACC_ROWS, _HALF), _f32),
            jax.ShapeDtypeStruct((_NC, _ACC_ROWS, 8), _f32),
        ],
        mesh=plsc.VectorSubcoreMesh(core_axis_name="c", subcore_axis_name="s"),
        compiler_params=pltpu.CompilerParams(use_tc_tiling_on_sc=False),
        scratch_types=[
            pltpu.VMEM((_CH_T, _CHUNK), _i32),
            pltpu.VMEM((_CH_T, _CHUNK), _i32),
            pltpu.VMEM((_CHUNK, _HALF), _f32),
            pltpu.VMEM((_CHUNK, _HALF), _f32),
            pltpu.VMEM((_CHUNK, _HALF), _f32),
            pltpu.VMEM((_CHUNK, _HALF), _f32),
            pltpu.VMEM((128, _HALF), _f32),
            pltpu.VMEM((128, 8), _f32),
            pltpu.VMEM((_CHUNK, 8), _f32),
            pltpu.SemaphoreType.DMA,
            pltpu.SemaphoreType.DMA,
            pltpu.SemaphoreType.DMA,
            pltpu.SemaphoreType.DMA,
            pltpu.VMEM_SHARED((_ACC_ROWS, _HALF), _f32),
            pltpu.VMEM_SHARED((_ACC_ROWS, 8), _f32),
            pltpu.VMEM_SHARED((232, _HALF), _f32),
        ],
    )


_GPW = 3 * _BATCH // _NW


def _sc_gather_body(table, idx, out, idx_v, rows_v, sem):
    c = lax.axis_index("c")
    s = lax.axis_index("s")
    wid = c * _NS + s
    pltpu.sync_copy(idx.at[wid], idx_v)
    pltpu.async_copy(table.at[idx_v], rows_v, sem).wait()
    pltpu.sync_copy(rows_v, out.at[pl.ds(wid * _GPW, _GPW)])


@functools.cache
def _get_sc_gather():
    return pl.kernel(
        _sc_gather_body,
        out_type=jax.ShapeDtypeStruct((3 * _BATCH, _HIDDEN), _f32),
        mesh=plsc.VectorSubcoreMesh(core_axis_name="c", subcore_axis_name="s"),
        scratch_types=[
            pltpu.VMEM((_GPW,), _i32),
            pltpu.VMEM((_GPW, _HIDDEN), _f32),
            pltpu.SemaphoreType.DMA,
        ],
    )


def _mm_body(x_ref, w_ref, o_ref):
    o_ref[...] = jnp.dot(x_ref[...], w_ref[...], preferred_element_type=_f32)


def _mm(x, w, bm):
    m, k = x.shape
    n = w.shape[1]
    return pl.pallas_call(
        _mm_body,
        grid=(m // bm,),
        in_specs=[
            pl.BlockSpec((bm, k), lambda i: (i, 0)),
            pl.BlockSpec((k, n), lambda i: (0, 0)),
        ],
        out_specs=pl.BlockSpec((bm, n), lambda i: (i, 0)),
        out_shape=jax.ShapeDtypeStruct((m, n), _f32),
    )(x, w)


def _split3(y, tbl_ref, hs_ref):
    tbl_ref[0] = y[:, :_HALF]
    tbl_ref[1] = y[:, _HALF:_HIDDEN]
    hs_ref[...] = y[:, _HIDDEN:]


def _mm_split_body(x_ref, w_ref, tbl_ref, hs_ref):
    _split3(jnp.dot(x_ref[...], w_ref[...], preferred_element_type=_f32),
            tbl_ref, hs_ref)


def _mm_split(x, w2, bm):
    m = x.shape[0]
    return pl.pallas_call(
        _mm_split_body,
        grid=(m // bm,),
        in_specs=[
            pl.BlockSpec((bm, _HIDDEN), lambda i: (i, 0)),
            pl.BlockSpec((_HIDDEN, 2 * _HIDDEN), lambda i: (0, 0)),
        ],
        out_specs=[
            pl.BlockSpec((_NC, bm, _HALF), lambda i: (0, i, 0)),
            pl.BlockSpec((bm, _HIDDEN), lambda i: (i, 0)),
        ],
        out_shape=[
            jax.ShapeDtypeStruct((_NC, m, _HALF), _f32),
            jax.ShapeDtypeStruct((m, _HIDDEN), _f32),
        ],
    )(x, w2)


def _fin_mm_body(a0_ref, a1_ref, d0_ref, d1_ref, hs_ref, w_ref,
                 tbl_ref, hs_next_ref):
    deg = jnp.maximum(d0_ref[0][:, 0:1] + d1_ref[0][:, 0:1], 1.0)
    agg = jnp.concatenate([a0_ref[0], a1_ref[0]], axis=1) / deg
    e = jnp.maximum(agg + hs_ref[...], 0.0)
    _split3(jnp.dot(e, w_ref[...], preferred_element_type=_f32),
            tbl_ref, hs_next_ref)


def _fin_mm(acc, degp, hs, w2, bm):
    m = hs.shape[0]
    return pl.pallas_call(
        _fin_mm_body,
        grid=(m // bm,),
        in_specs=[
            pl.BlockSpec((1, bm, _HALF), lambda i: (0, i, 0)),
            pl.BlockSpec((1, bm, _HALF), lambda i: (1, i, 0)),
            pl.BlockSpec((1, bm, 8), lambda i: (0, i, 0)),
            pl.BlockSpec((1, bm, 8), lambda i: (1, i, 0)),
            pl.BlockSpec((bm, _HIDDEN), lambda i: (i, 0)),
            pl.BlockSpec((_HIDDEN, 2 * _HIDDEN), lambda i: (0, 0)),
        ],
        out_specs=[
            pl.BlockSpec((_NC, bm, _HALF), lambda i: (0, i, 0)),
            pl.BlockSpec((bm, _HIDDEN), lambda i: (i, 0)),
        ],
        out_shape=[
            jax.ShapeDtypeStruct((_NC, m, _HALF), _f32),
            jax.ShapeDtypeStruct((m, _HIDDEN), _f32),
        ],
    )(acc, acc, degp, degp, hs, w2)


def _fin_body(a0_ref, a1_ref, d0_ref, d1_ref, hs_ref, o_ref):
    deg = jnp.maximum(d0_ref[...][:, 0:1] + d1_ref[...][:, 0:1], 1.0)
    agg = jnp.concatenate([a0_ref[...], a1_ref[...]], axis=1) / deg
    o_ref[...] = jnp.maximum(agg + hs_ref[...], 0.0)


def _finalize(a0, a1, d0, d1, hs, bm):
    m = a0.shape[0]
    return pl.pallas_call(
        _fin_body,
        grid=(m // bm,),
        in_specs=[
            pl.BlockSpec((bm, _HALF), lambda i: (i, 0)),
            pl.BlockSpec((bm, _HALF), lambda i: (i, 0)),
            pl.BlockSpec((bm, 8), lambda i: (i, 0)),
            pl.BlockSpec((bm, 8), lambda i: (i, 0)),
            pl.BlockSpec((bm, _HIDDEN), lambda i: (i, 0)),
        ],
        out_specs=pl.BlockSpec((bm, _HIDDEN), lambda i: (i, 0)),
        out_shape=jax.ShapeDtypeStruct((m, _HIDDEN), _f32),
    )(a0, a1, d0, d1, hs)


def _gru_body(x_ref, h_ref, wih_ref, whh_ref, bih_ref, bhh_ref, wm_ref,
              o_ref, rw0_ref, rw1_ref):
    x = x_ref[...]
    h = h_ref[...]
    gi = jnp.dot(x, wih_ref[...], preferred_element_type=_f32) + bih_ref[...]
    gh = jnp.dot(h, whh_ref[...], preferred_element_type=_f32) + bhh_ref[...]
    hd = _HIDDEN
    r = jax.nn.sigmoid(gi[:, :hd] + gh[:, :hd])
    z = jax.nn.sigmoid(gi[:, hd:2 * hd] + gh[:, hd:2 * hd])
    n = jnp.tanh(gi[:, 2 * hd:] + r * gh[:, 2 * hd:])
    hn = (1.0 - z) * n + z * h
    norm = jnp.sqrt(jnp.sum(hn * hn, axis=1, keepdims=True))
    relh = hn / jnp.maximum(norm, 1e-12)
    o_ref[...] = relh
    rwc = jnp.dot(relh, wm_ref[...], preferred_element_type=_f32)
    rw0_ref[0] = rwc[:, :_HALF]
    rw0_ref[1] = rwc[:, _HALF:_HIDDEN]
    rw1_ref[0] = rwc[:, _HIDDEN:_HIDDEN + _HALF]
    rw1_ref[1] = rwc[:, _HIDDEN + _HALF:]


def _gru(x, h, wih_t, whh_t, bih, bhh, wm01):
    m = x.shape[0]
    args = (x, h, wih_t, whh_t, bih, bhh, wm01)
    return pl.pallas_call(
        _gru_body,
        in_specs=[pl.BlockSpec(a.shape, lambda: (0,) * a.ndim) for a in args],
        out_specs=[
            pl.BlockSpec((m, _HIDDEN), lambda: (0, 0)),
            pl.BlockSpec((_NC, m, _HALF), lambda: (0, 0, 0)),
            pl.BlockSpec((_NC, m, _HALF), lambda: (0, 0, 0)),
        ],
        out_shape=[
            jax.ShapeDtypeStruct((m, _HIDDEN), _f32),
            jax.ShapeDtypeStruct((_NC, m, _HALF), _f32),
            jax.ShapeDtypeStruct((_NC, m, _HALF), _f32),
        ],
    )(*args)


def _dec_body(e1_ref, e2_ref, w_ref, b_ref, fcw_ref, fcb_ref, o_ref, *, bm):
    e1 = e1_ref[...]
    e2 = e2_ref[...]
    z = jnp.zeros((bm, 1), _f32)
    shifts = []
    for e in (e1, e2):
        shifts.append(jnp.concatenate([z, e[:, :-1]], axis=1))
        shifts.append(e)
        shifts.append(jnp.concatenate([e[:, 1:], z], axis=1))
    acc = jnp.zeros((bm, _HIDDEN), _f32)
    for ch in range(50):
        conv = b_ref[ch]
        for j in range(6):
            conv = conv + shifts[j] * w_ref[6 * ch + j]
        conv = jnp.maximum(conv, 0.0)
        acc = acc + jnp.dot(conv, fcw_ref[ch], preferred_element_type=_f32)
    o_ref[...] = jnp.maximum(acc + fcb_ref[...], 0.0)


def _decoder(e1, e2, conv_w, conv_b, fc_w, fc_b, bm=256):
    m = e1.shape[0]
    wflat = conv_w.reshape(300)
    fcw3 = fc_w.reshape(50, _HIDDEN, _HIDDEN)
    fcb = fc_b.reshape(1, _HIDDEN)
    return pl.pallas_call(
        functools.partial(_dec_body, bm=bm),
        grid=(m // bm,),
        in_specs=[
            pl.BlockSpec((bm, _HIDDEN), lambda i: (i, 0)),
            pl.BlockSpec((bm, _HIDDEN), lambda i: (i, 0)),
            pl.BlockSpec(memory_space=pltpu.SMEM),
            pl.BlockSpec(memory_space=pltpu.SMEM),
            pl.BlockSpec((50, _HIDDEN, _HIDDEN), lambda i: (0, 0, 0)),
            pl.BlockSpec((1, _HIDDEN), lambda i: (0, 0)),
        ],
        out_specs=pl.BlockSpec((bm, _HIDDEN), lambda i: (i, 0)),
        out_shape=jax.ShapeDtypeStruct((m, _HIDDEN), _f32),
    )(e1, e2, wflat, conv_b, fcw3, fcb)


def _score_body(y_ref, t_ref, o_ref):
    o_ref[...] = lax.dot_general(
        y_ref[...], t_ref[...], (((1,), (1,)), ((), ())),
        preferred_element_type=_f32)


def _score(y, table, bn):
    m = y.shape[0]
    n = table.shape[0]
    return pl.pallas_call(
        _score_body,
        grid=(n // bn,),
        in_specs=[
            pl.BlockSpec((m, _HIDDEN), lambda i: (0, 0)),
            pl.BlockSpec((bn, _HIDDEN), lambda i: (i, 0)),
        ],
        out_specs=pl.BlockSpec((m, bn), lambda i: (0, i)),
        out_shape=jax.ShapeDtypeStruct((m, n), _f32),
    )(y, table)


def _pad_to(x, n, val):
    return jnp.concatenate(
        [x, jnp.full((n - x.shape[0],), val, dtype=x.dtype)])


@jax.jit
def kernel(ent_embeds, rel_embeds, time_embeds, rgcn_w_msg, rgcn_w_self,
           gru_w_ih, gru_w_hh, gru_b_ih, gru_b_hh, conve_w, conve_b,
           conve_fc_w, conve_fc_b, convr_w, convr_b, convr_fc_w, convr_fc_b,
           edge_src, edge_dst, edge_type, subj, rel, obj):
    hist = time_embeds.shape[0]
    nlayers = rgcn_w_msg.shape[0]

    src = edge_src.astype(_i32)
    dst = edge_dst.astype(_i32)
    ety = edge_type.astype(_i32)

    gidx_ts, sdst_ts = [], []
    for t in range(hist):
        sh3 = (_NS, _CH_T // 2, _CHUNK)
        ga = _pad_to(src[t], _PART, 0).reshape(sh3)
        gb = _pad_to(ety[t], _PART, 0).reshape(sh3)
        da = _pad_to(dst[t], _PART, _TRASH).reshape(sh3)
        gidx_ts.append(jnp.concatenate([ga, gb], axis=1))
        sdst_ts.append(jnp.concatenate([da, da], axis=1))

    z64 = jnp.zeros((128, _HALF), _f32)
    z16 = jnp.zeros((128, 8), _f32)
    ones16 = jnp.ones((_CHUNK, 8), _f32)

    wih_t = gru_w_ih.T
    whh_t = gru_w_hh.T
    bih = gru_b_ih.reshape(1, -1)
    bhh = gru_b_hh.reshape(1, -1)
    relpad = jnp.zeros((232 - _NUM_RELS, _HIDDEN), _f32)
    rel_p = jnp.concatenate([rel_embeds, relpad])

    w2s = [jnp.concatenate([rgcn_w_msg[l], rgcn_w_self[l]], axis=1)
           for l in range(nlayers)]
    wm01 = jnp.concatenate([rgcn_w_msg[0], rgcn_w_msg[1]], axis=1)

    tbl, hs = _mm_split(ent_embeds, w2s[0], bm=2000)
    rwc0 = _mm(rel_p, wm01, bm=232)
    rw0 = rwc0[:, :_HIDDEN].reshape(232, _NC, _HALF).transpose(1, 0, 2)
    rw1 = rwc0[:, _HIDDEN:].reshape(232, _NC, _HALF).transpose(1, 0, 2)

    relh_p = rel_p
    for t in range(hist):
        rw_t = (rw0, rw1)
        for l in range(nlayers):
            acc, degp = _get_sc_rgcn()(tbl, rw_t[l], gidx_ts[t], sdst_ts[t],
                                       z64, z16, ones16)
            if t == hist - 1 and l == nlayers - 1:
                ent = _finalize(acc[0, :_NUM_ENTS], acc[1, :_NUM_ENTS],
                                degp[0, :_NUM_ENTS], degp[1, :_NUM_ENTS],
                                hs, bm=2000)
            else:
                tbl, hs = _fin_mm(acc, degp, hs, w2s[(l + 1) % nlayers],
                                  bm=2000)
        relh_p, rw0, rw1 = _gru(rel_p, relh_p, wih_t, whh_t, bih, bhh, wm01)

    relh = relh_p[:_NUM_RELS]

    table2 = jnp.concatenate([ent, relh])
    idx = jnp.concatenate([subj.astype(_i32), obj.astype(_i32),
                           rel.astype(_i32) + _NUM_ENTS]).reshape(_NW, _GPW)
    rows = _get_sc_gather()(table2, idx)
    e_subj = rows[:_BATCH]
    e_obj = rows[_BATCH:2 * _BATCH]
    e_rel = rows[2 * _BATCH:]

    y1 = _decoder(e_subj, e_rel, conve_w, conve_b, conve_fc_w, conve_fc_b)
    y2 = _decoder(e_subj, e_obj, convr_w, convr_b, convr_fc_w, convr_fc_b)

    ent10240 = jnp.concatenate([ent, jnp.zeros((240, _HIDDEN), _f32)])
    ent_logit = _score(y1, ent10240, bn=2048)[:, :_NUM_ENTS]
    relh256 = jnp.concatenate([relh, jnp.zeros((26, _HIDDEN), _f32)])
    rel_logit = _score(y2, relh256, bn=256)[:, :_NUM_RELS]
    return ent_logit, rel_logit
